# Initial kernel scaffold; baseline (speedup 1.0000x reference)
import jax
import jax.numpy as jnp
from jax.experimental import pallas as pl


def kernel(node_feature, adapt_W, adapt_b, k_W, k_b, q_W, q_b, v_W, v_b, a_W, a_b, rel_pri, rel_att, rel_msg, skip, rte_W, rte_b, rte_emb, node_type, edge_time, edge_index, edge_type):
    raise NotImplementedError("write your pallas kernel here")


# speedup vs baseline: 4.3858x; 4.3858x over previous
import functools
import math

import jax
import jax.numpy as jnp
from jax import lax
from jax.experimental import pallas as pl
from jax.experimental.pallas import tpu as pltpu
from jax.experimental.pallas import tpu_sc as plsc

N = 10000
E = 320000
IN_DIM = 128
HID = 128
T = 4
R = 8
H = 8
DK = HID // H
L = 2
ML = 240
SQRT_DK = math.sqrt(DK)

NC = 2
NS = 16
NW = NC * NS
CB = 128
CB2 = 32
CB1B = 512
NB = 1000
GRID = N // NB
NEG = -3.0e38


def _f32(*shape):
    return jax.ShapeDtypeStruct(shape, jnp.float32)


def _i32(*shape):
    return jax.ShapeDtypeStruct(shape, jnp.int32)


def _adapt_body(nf, oh, W, b, o):
    x = nf[:]
    acc = jnp.zeros((NB, HID), jnp.float32)
    for t in range(T):
        y = jnp.dot(x, W[t], preferred_element_type=jnp.float32) + b[t][None, :]
        acc = acc + oh[:, t][:, None] * y
    o[:] = jnp.tanh(acc)


def _adapt(nf, oh, W, b):
    return pl.pallas_call(
        _adapt_body,
        grid=(GRID,),
        in_specs=[
            pl.BlockSpec((NB, IN_DIM), lambda i: (i, 0)),
            pl.BlockSpec((NB, T), lambda i: (i, 0)),
            pl.BlockSpec((T, IN_DIM, HID), lambda i: (0, 0, 0)),
            pl.BlockSpec((T, HID), lambda i: (0, 0)),
        ],
        out_specs=pl.BlockSpec((NB, HID), lambda i: (i, 0)),
        out_shape=_f32(N, HID),
    )(nf, oh, W, b)


def _ptl_block(x, oh, W, b):
    acc = jnp.zeros((x.shape[0], HID), jnp.float32)
    for t in range(T):
        y = jnp.dot(x, W[t], preferred_element_type=jnp.float32) + b[t][None, :]
        acc = acc + oh[:, t][:, None] * y
    return acc


def _proj_body(x, oh, kW, kb, qW, qb, vW, vb, BDk, BDv, qo, ko, vo):
    xb = x[:]
    ohb = oh[:]
    Kn = _ptl_block(xb, ohb, kW, kb)
    qo[:] = _ptl_block(xb, ohb, qW, qb)
    Vn = _ptl_block(xb, ohb, vW, vb)
    for r in range(R):
        ko[:, r * HID:(r + 1) * HID] = jnp.dot(
            Kn, BDk[r], preferred_element_type=jnp.float32)
        vo[:, r * HID:(r + 1) * HID] = jnp.dot(
            Vn, BDv[r], preferred_element_type=jnp.float32)


def _proj(x, oh, kW, kb, qW, qb, vW, vb, BDk, BDv):
    wspec3 = pl.BlockSpec((T, HID, HID), lambda i: (0, 0, 0))
    wspec2 = pl.BlockSpec((T, HID), lambda i: (0, 0))
    bdspec = pl.BlockSpec((R, HID, HID), lambda i: (0, 0, 0))
    return pl.pallas_call(
        _proj_body,
        grid=(GRID,),
        in_specs=[
            pl.BlockSpec((NB, HID), lambda i: (i, 0)),
            pl.BlockSpec((NB, T), lambda i: (i, 0)),
            wspec3, wspec2, wspec3, wspec2, wspec3, wspec2, bdspec, bdspec,
        ],
        out_specs=[
            pl.BlockSpec((NB, HID), lambda i: (i, 0)),
            pl.BlockSpec((NB, R * HID), lambda i: (i, 0)),
            pl.BlockSpec((NB, R * HID), lambda i: (i, 0)),
        ],
        out_shape=[_f32(N, HID), _f32(N, R * HID), _f32(N, R * HID)],
    )(x, oh, kW, kb, qW, qb, vW, vb, BDk, BDv)


def _rk2_body(emb, rteW, rteb, kW, vW, BDk, BDv, ko, vo):
    r_vec = jnp.dot(emb[:], rteW[:],
                    preferred_element_type=jnp.float32) + rteb[:]
    for t in range(T):
        RKt = jnp.dot(r_vec, kW[t], preferred_element_type=jnp.float32)
        RVt = jnp.dot(r_vec, vW[t], preferred_element_type=jnp.float32)
        for r in range(R):
            c = (t * R + r) * HID
            ko[:, c:c + HID] = jnp.dot(RKt, BDk[r],
                                       preferred_element_type=jnp.float32)
            vo[:, c:c + HID] = jnp.dot(RVt, BDv[r],
                                       preferred_element_type=jnp.float32)


def _rk2(emb, rteW, rteb, kW, vW, BDk, BDv):
    return pl.pallas_call(
        _rk2_body,
        out_shape=[_f32(ML, T * R * HID), _f32(ML, T * R * HID)],
    )(emb, rteW, rteb.reshape(1, HID), kW, vW, BDk, BDv)


def _merge_body(p, o):
    o[:] = jnp.max(p[:], axis=0)


def _merge(mpart):
    return pl.pallas_call(
        _merge_body,
        out_shape=_f32(N * H),
    )(mpart)


def _combine_body(final, aggp, sp, x, oh, aW, ab, skp, REP, o):
    agg = aggp[0] + aggp[1]
    s16 = sp[0] + sp[1]
    denom = jnp.dot(s16, REP[:], preferred_element_type=jnp.float32) + 1e-16
    aggr = agg / denom
    aggr = 0.5 * aggr * (1.0 + lax.erf(aggr / math.sqrt(2.0)))
    trans = _ptl_block(aggr, oh[:], aW, ab)
    alphas = jax.nn.sigmoid(skp[:])
    alpha = jnp.sum(oh[:] * alphas, axis=1, keepdims=True)
    y = trans * alpha + x[:] * (1.0 - alpha)
    if final:
        y = y / jnp.sqrt(jnp.sum(y * y, axis=-1, keepdims=True))
    o[:] = y


def _combine(aggp, sp, x, oh, aW, ab, skp, final):
    return pl.pallas_call(
        functools.partial(_combine_body, final),
        grid=(GRID,),
        in_specs=[
            pl.BlockSpec((NC, NB, HID), lambda i: (0, i, 0)),
            pl.BlockSpec((NC, NB, 16), lambda i: (0, i, 0)),
            pl.BlockSpec((NB, HID), lambda i: (i, 0)),
            pl.BlockSpec((NB, T), lambda i: (i, 0)),
            pl.BlockSpec((T, HID, HID), lambda i: (0, 0, 0)),
            pl.BlockSpec((T, HID), lambda i: (0, 0)),
            pl.BlockSpec((1, T), lambda i: (0, 0)),
            pl.BlockSpec((16, HID), lambda i: (0, 0)),
        ],
        out_specs=pl.BlockSpec((NB, HID), lambda i: (i, 0)),
        out_shape=_f32(N, HID),
    )(aggp, sp, x, oh, aW, ab, skp.reshape(1, T), _rep_matrix())


def _rep_matrix():
    i = jnp.arange(16)[:, None]
    j = jnp.arange(HID)[None, :]
    return jnp.where((j // DK) == i, 1.0, 0.0).astype(jnp.float32)


def _block_diag(A):
    r = A.shape[0]
    out = jnp.zeros((r, H, DK, H, DK), A.dtype)
    idx = jnp.arange(H)
    out = out.at[:, idx, :, idx, :].set(jnp.moveaxis(A, 1, 0))
    return out.reshape(r, HID, HID)


_SC_PARAMS = pltpu.CompilerParams(
    needs_layout_passes=False, use_tc_tiling_on_sc=False)


@functools.cache
def _mesh():
    return plsc.VectorSubcoreMesh(core_axis_name="c", subcore_axis_name="s",
                                  num_cores=NC, num_subcores=NS)


def _worker_id():
    return lax.axis_index("c") * NS + lax.axis_index("s")


def _nj(nchunk):
    w = _worker_id()
    extra = nchunk - (nchunk // NW) * NW
    return w, jnp.where(w < extra, nchunk // NW + 1, nchunk // NW)


def _iota16():
    return lax.iota(jnp.int32, 16)


def _pipeline(nchunk, load_issue, wait_compute):
    w, nj = _nj(nchunk)

    def ci(i):
        return w + NW * i

    load_issue(ci(0), 0)

    def pair(j, carry):
        i1 = 2 * j + 1
        i2 = 2 * j + 2

        @pl.when(i1 < nj)
        def _():
            load_issue(ci(i1), 1)
        wait_compute(ci(2 * j), 0)

        @pl.when(i2 < nj)
        def _():
            load_issue(ci(i2), 0)

        @pl.when(i1 < nj)
        def _():
            wait_compute(ci(i1), 1)
        return carry

    lax.fori_loop(0, (nj + 1) // 2, pair, 0)


def _pass1a_body(first, *refs):
    if first:
        (qn, kall, rk2, epack, ntr,
         att_o, eout_o,
         ntv, eb0, eb1, ob0, ob1,
         q0, k0, rk0, q1, k1, rk1, attT,
         sq0, sk0, sr0, sq1, sk1, sr1) = refs
        ebufs = (eb0, eb1)
        obufs = (ob0, ob1)
        pltpu.sync_copy(ntr, ntv.at[pl.ds(0, N)])
    else:
        (qn, kall, rk2, epack,
         att_o,
         eb0, eb1,
         q0, k0, rk0, q1, k1, rk1, attT,
         sq0, sk0, sr0, sq1, sk1, sr1) = refs
        ebufs = (eb0, eb1)
        obufs = ebufs
    qb = (q0, q1)
    kb = (k0, k1)
    rkb = (rk0, rk1)
    sems = ((sq0, sk0, sr0), (sq1, sk1, sr1))

    def load_issue(c, b):
        base = c * CB
        eb = ebufs[b]
        ob = obufs[b]
        pltpu.sync_copy(epack.at[:, pl.ds(base, CB)], eb)
        if first:
            for g in range(CB // 16):
                sl = pl.ds(g * 16, 16)
                s16 = eb[0, sl]
                d16 = eb[1, sl]
                e16 = eb[2, sl]
                r16 = eb[3, sl]
                tj = plsc.load_gather(ntv, [s16])
                ob[0, sl] = d16
                ob[1, sl] = s16 * R + r16
                ob[2, sl] = (e16 * T + tj) * R + r16
                ob[3, sl] = d16
            pltpu.sync_copy(ob, eout_o.at[:, pl.ds(base, CB)])
        pltpu.async_copy(qn.at[ob.at[0]], qb[b], sems[b][0])
        pltpu.async_copy(kall.at[ob.at[1]], kb[b], sems[b][1])
        pltpu.async_copy(rk2.at[ob.at[2]], rkb[b], sems[b][2])

    def wait_compute(c, b):
        base = c * CB
        ob = obufs[b]
        pltpu.make_async_copy(qn.at[ob.at[0]], qb[b], sems[b][0]).wait()
        pltpu.make_async_copy(kall.at[ob.at[1]], kb[b], sems[b][1]).wait()
        pltpu.make_async_copy(rk2.at[ob.at[2]], rkb[b], sems[b][2]).wait()

        def grp(g, carry):
            erow = g * 16 + _iota16()
            for h in range(H):
                acc = jnp.zeros((16,), jnp.float32)
                for d in range(DK):
                    col = jnp.full((16,), h * DK + d, jnp.int32)
                    qv = plsc.load_gather(qb[b], [erow, col])
                    kv = plsc.load_gather(kb[b], [erow, col])
                    rv = plsc.load_gather(rkb[b], [erow, col])
                    acc = acc + qv * (kv + rv)
                attT[h, pl.ds(g * 16, 16)] = acc
            return carry
        lax.fori_loop(0, CB // 16, grp, 0)
        pltpu.sync_copy(attT, att_o.at[:, pl.ds(base, CB)])

    _pipeline(E // CB, load_issue, wait_compute)


def _row_bufs(cb):
    return [pltpu.VMEM((cb, HID), jnp.float32)] * 3


def _pass1a_first(Qn, Kall, RK2, epack, nt):
    scratch = (
        [pltpu.VMEM((10112,), jnp.int32)]
        + [pltpu.VMEM((4, CB), jnp.int32)] * 4
        + _row_bufs(CB) + _row_bufs(CB)
        + [pltpu.VMEM((H, CB), jnp.float32)]
        + [pltpu.SemaphoreType.DMA] * 6
    )
    fn = pl.kernel(
        functools.partial(_pass1a_body, True),
        out_type=(_f32(H, E), _i32(4, E)),
        mesh=_mesh(),
        scratch_types=scratch,
        compiler_params=_SC_PARAMS,
    )
    return fn(Qn, Kall, RK2, epack, nt)


def _pass1a_rest(Qn, Kall, RK2, eout):
    scratch = (
        [pltpu.VMEM((4, CB), jnp.int32)] * 2
        + _row_bufs(CB) + _row_bufs(CB)
        + [pltpu.VMEM((H, CB), jnp.float32)]
        + [pltpu.SemaphoreType.DMA] * 6
    )
    fn = pl.kernel(
        functools.partial(_pass1a_body, False),
        out_type=_f32(H, E),
        mesh=_mesh(),
        scratch_types=scratch,
        compiler_params=_SC_PARAMS,
    )
    return fn(Qn, Kall, RK2, eout)


def _pass1b_body(attr, eout, mpart_o, dstv, attT, mloc):
    neg = jnp.full((16,), NEG, jnp.float32)

    def init(i, c):
        mloc[pl.ds(i * 16, 16)] = neg
        return c
    lax.fori_loop(0, (N * H) // 16, init, 0)

    w, nj = _nj(E // CB1B)

    def chunk(i, carry):
        base = (w + NW * i) * CB1B
        pltpu.sync_copy(eout.at[0, pl.ds(base, CB1B)], dstv)
        pltpu.sync_copy(attr.at[:, pl.ds(base, CB1B)], attT)

        def grp(g, carry2):
            dst16 = dstv[pl.ds(g * 16, 16)]
            for h in range(H):
                idx = dst16 * H + h
                val = attT[h, pl.ds(g * 16, 16)]
                cur = plsc.load_gather(mloc, [idx])
                msk = val > cur

                def cond(mm):
                    return jnp.any(mm)

                def body(mm):
                    plsc.store_scatter(mloc, [idx], val, mask=mm)
                    cur2 = plsc.load_gather(mloc, [idx])
                    return mm & (val > cur2)
                lax.while_loop(cond, body, msk)
            return carry2
        lax.fori_loop(0, CB1B // 16, grp, 0)
        return carry
    lax.fori_loop(0, nj, chunk, 0)
    pltpu.sync_copy(mloc, mpart_o.at[_worker_id()])


def _pass1b(att, eout):
    scratch = [
        pltpu.VMEM((CB1B,), jnp.int32),
        pltpu.VMEM((H, CB1B), jnp.float32),
        pltpu.VMEM((N * H,), jnp.float32),
    ]
    fn = pl.kernel(
        _pass1b_body,
        out_type=_f32(NW, N * H),
        mesh=_mesh(),
        scratch_types=scratch,
        compiler_params=_SC_PARAMS,
    )
    return fn(att, eout)


ROWS_PER_TILE = N // NS
ZCH = 5


def _pass2_body(attr, mr, eout, vall, rv2,
                aggp_o, sp_o,
                eb0, eb1, m0, m1, v0, rv0, v1, rv1, a0, a1,
                msgrows, wT, wrows, zbuf, zs, agg_s, s_s,
                sm0, sv0, sr0, sm1, sv1, sr1):
    cid = lax.axis_index("c")
    sid = lax.axis_index("s")
    r0 = sid * ROWS_PER_TILE
    ebufs = (eb0, eb1)
    mb = (m0, m1)
    vb = (v0, v1)
    rvb = (rv0, rv1)
    ab = (a0, a1)
    sems = ((sm0, sv0, sr0), (sm1, sv1, sr1))

    z16 = jnp.zeros((16,), jnp.float32)

    def zinit(i, c):
        for cc in range(HID // 16):
            zbuf[i, pl.ds(cc * 16, 16)] = z16
        zs[i, :] = z16
        return c
    lax.fori_loop(0, ZCH, zinit, 0)

    def winit(i, c):
        wrows[i, :] = z16
        return c
    lax.fori_loop(0, CB2, winit, 0)

    for j in range(ROWS_PER_TILE // ZCH):
        pltpu.sync_copy(zbuf, agg_s.at[pl.ds(r0 + j * ZCH, ZCH)])
        pltpu.sync_copy(zs, s_s.at[pl.ds(r0 + j * ZCH, ZCH)])
    plsc.subcore_barrier()

    def load_issue(c, b):
        base = c * CB2
        eb = ebufs[b]
        pltpu.sync_copy(eout.at[:, pl.ds(base, CB2)], eb)
        pltpu.sync_copy(attr.at[:, pl.ds(base, CB2)], ab[b])
        pltpu.async_copy(mr.at[eb.at[0]], mb[b], sems[b][0])
        pltpu.async_copy(vall.at[eb.at[1]], vb[b], sems[b][1])
        pltpu.async_copy(rv2.at[eb.at[2]], rvb[b], sems[b][2])

    def wait_compute(c, b):
        eb = ebufs[b]
        pltpu.make_async_copy(mr.at[eb.at[0]], mb[b], sems[b][0]).wait()
        pltpu.make_async_copy(vall.at[eb.at[1]], vb[b], sems[b][1]).wait()
        pltpu.make_async_copy(rv2.at[eb.at[2]], rvb[b], sems[b][2]).wait()

        for g in range(CB2 // 16):
            erow = g * 16 + _iota16()
            for h in range(H):
                hcol = jnp.full((16,), h, jnp.int32)
                m16 = plsc.load_gather(mb[b], [erow, hcol])
                a16 = ab[b][h, pl.ds(g * 16, 16)]
                w16 = jnp.exp(a16 - m16)
                wT[h, pl.ds(g * 16, 16)] = w16
                plsc.store_scatter(wrows, [erow, hcol], w16)

        def edge(e, carry):
            e16 = jnp.full((16,), e, jnp.int32)
            for h in range(H):
                wb = plsc.load_gather(wT, [jnp.full((16,), h, jnp.int32), e16])
                v16 = vb[b][e, pl.ds(h * DK, DK)]
                rv16 = rvb[b][e, pl.ds(h * DK, DK)]
                msgrows[e, pl.ds(h * DK, DK)] = wb * (v16 + rv16)
            return carry
        lax.fori_loop(0, CB2, edge, 0)

        pltpu.sync_copy(msgrows, agg_s.at[eb.at[0]], add=True)
        pltpu.sync_copy(wrows, s_s.at[eb.at[0]], add=True)

    _pipeline(E // CB2, load_issue, wait_compute)
    plsc.subcore_barrier()

    for j in range(ROWS_PER_TILE // ZCH):
        pltpu.sync_copy(agg_s.at[pl.ds(r0 + j * ZCH, ZCH)], zbuf)
        pltpu.sync_copy(zbuf, aggp_o.at[cid, pl.ds(r0 + j * ZCH, ZCH)])
        pltpu.sync_copy(s_s.at[pl.ds(r0 + j * ZCH, ZCH)], zs)
        pltpu.sync_copy(zs, sp_o.at[cid, pl.ds(r0 + j * ZCH, ZCH)])


def _pass2(att, m2, eout, Vall, RV2):
    scratch = [
        pltpu.VMEM((4, CB2), jnp.int32),
        pltpu.VMEM((4, CB2), jnp.int32),
        pltpu.VMEM((CB2, 16), jnp.float32),
        pltpu.VMEM((CB2, 16), jnp.float32),
        pltpu.VMEM((CB2, HID), jnp.float32),
        pltpu.VMEM((CB2, HID), jnp.float32),
        pltpu.VMEM((CB2, HID), jnp.float32),
        pltpu.VMEM((CB2, HID), jnp.float32),
        pltpu.VMEM((H, CB2), jnp.float32),
        pltpu.VMEM((H, CB2), jnp.float32),
        pltpu.VMEM((CB2, HID), jnp.float32),
        pltpu.VMEM((H, CB2), jnp.float32),
        pltpu.VMEM((CB2, 16), jnp.float32),
        pltpu.VMEM((ZCH, HID), jnp.float32),
        pltpu.VMEM((ZCH, 16), jnp.float32),
        pltpu.VMEM_SHARED((N, HID), jnp.float32),
        pltpu.VMEM_SHARED((N, 16), jnp.float32),
    ] + [pltpu.SemaphoreType.DMA] * 6
    fn = pl.kernel(
        _pass2_body,
        out_type=(_f32(NC, N, HID), _f32(NC, N, 16)),
        mesh=_mesh(),
        scratch_types=scratch,
        compiler_params=_SC_PARAMS,
    )
    return fn(att, m2, eout, Vall, RV2)


def kernel(node_feature, adapt_W, adapt_b, k_W, k_b, q_W, q_b, v_W, v_b,
           a_W, a_b, rel_pri, rel_att, rel_msg, skip, rte_W, rte_b,
           rte_emb, node_type, edge_time, edge_index, edge_type):
    nt = node_type.astype(jnp.int32)
    epack = jnp.stack([edge_index[0].astype(jnp.int32),
                       edge_index[1].astype(jnp.int32),
                       edge_time.astype(jnp.int32),
                       edge_type.astype(jnp.int32)])
    oh = jax.nn.one_hot(nt, T, dtype=jnp.float32)

    x = _adapt(node_feature, oh, adapt_W, adapt_b)
    eout = None
    for l in range(L):
        scale = rel_pri[l] / SQRT_DK
        BDk = _block_diag(rel_att[l] * scale[..., None, None])
        BDv = _block_diag(rel_msg[l])
        Qn, Kall, Vall = _proj(x, oh, k_W[l], k_b[l], q_W[l], q_b[l],
                               v_W[l], v_b[l], BDk, BDv)
        RK2, RV2 = _rk2(rte_emb, rte_W[l], rte_b[l], k_W[l], v_W[l], BDk, BDv)
        Kall = Kall.reshape(N * R, HID)
        Vall = Vall.reshape(N * R, HID)
        RK2 = RK2.reshape(ML * T * R, HID)
        RV2 = RV2.reshape(ML * T * R, HID)
        if l == 0:
            att, eout = _pass1a_first(Qn, Kall, RK2, epack, nt)
        else:
            att = _pass1a_rest(Qn, Kall, RK2, eout)
        mpart = _pass1b(att, eout)
        m = _merge(mpart)
        m2 = jnp.pad(m.reshape(N, H), ((0, 0), (0, 16 - H)))
        aggp, sp = _pass2(att, m2, eout, Vall, RV2)
        x = _combine(aggp, sp, x, oh, a_W[l], a_b[l], skip[l],
                     final=(l == L - 1))
    return x


# speedup vs baseline: 5.9345x; 1.3531x over previous
import functools
import math

import jax
import jax.numpy as jnp
from jax import lax
from jax.experimental import pallas as pl
from jax.experimental.pallas import tpu as pltpu
from jax.experimental.pallas import tpu_sc as plsc

N = 10000
E = 320000
IN_DIM = 128
HID = 128
T = 4
R = 8
H = 8
DK = HID // H
L = 2
ML = 240
SQRT_DK = math.sqrt(DK)

NC = 2
NS = 16
NW = NC * NS
CB = 128
CB2 = 32
CB1B = 512
NB = 1000
GRID = N // NB
NEG = -3.0e38


def _f32(*shape):
    return jax.ShapeDtypeStruct(shape, jnp.float32)


def _i32(*shape):
    return jax.ShapeDtypeStruct(shape, jnp.int32)


def _adapt_body(nf, oh, W, b, o):
    x = nf[:]
    acc = jnp.zeros((NB, HID), jnp.float32)
    for t in range(T):
        y = jnp.dot(x, W[t], preferred_element_type=jnp.float32) + b[t][None, :]
        acc = acc + oh[:, t][:, None] * y
    o[:] = jnp.tanh(acc)


def _adapt(nf, oh, W, b):
    return pl.pallas_call(
        _adapt_body,
        grid=(GRID,),
        in_specs=[
            pl.BlockSpec((NB, IN_DIM), lambda i: (i, 0)),
            pl.BlockSpec((NB, T), lambda i: (i, 0)),
            pl.BlockSpec((T, IN_DIM, HID), lambda i: (0, 0, 0)),
            pl.BlockSpec((T, HID), lambda i: (0, 0)),
        ],
        out_specs=pl.BlockSpec((NB, HID), lambda i: (i, 0)),
        out_shape=_f32(N, HID),
    )(nf, oh, W, b)


def _ptl_block(x, oh, W, b):
    acc = jnp.zeros((x.shape[0], HID), jnp.float32)
    for t in range(T):
        y = jnp.dot(x, W[t], preferred_element_type=jnp.float32) + b[t][None, :]
        acc = acc + oh[:, t][:, None] * y
    return acc


def _proj_body(x, oh, kW, kb, qW, qb, vW, vb, BDk, BDv, qo, ko, vo):
    xb = x[:]
    ohb = oh[:]
    Kn = _ptl_block(xb, ohb, kW, kb)
    qo[:] = _ptl_block(xb, ohb, qW, qb)
    Vn = _ptl_block(xb, ohb, vW, vb)
    for r in range(R):
        ko[:, r * HID:(r + 1) * HID] = jnp.dot(
            Kn, BDk[r], preferred_element_type=jnp.float32)
        vo[:, r * HID:(r + 1) * HID] = jnp.dot(
            Vn, BDv[r], preferred_element_type=jnp.float32)


def _proj(x, oh, kW, kb, qW, qb, vW, vb, BDk, BDv):
    wspec3 = pl.BlockSpec((T, HID, HID), lambda i: (0, 0, 0))
    wspec2 = pl.BlockSpec((T, HID), lambda i: (0, 0))
    bdspec = pl.BlockSpec((R, HID, HID), lambda i: (0, 0, 0))
    return pl.pallas_call(
        _proj_body,
        grid=(GRID,),
        in_specs=[
            pl.BlockSpec((NB, HID), lambda i: (i, 0)),
            pl.BlockSpec((NB, T), lambda i: (i, 0)),
            wspec3, wspec2, wspec3, wspec2, wspec3, wspec2, bdspec, bdspec,
        ],
        out_specs=[
            pl.BlockSpec((NB, HID), lambda i: (i, 0)),
            pl.BlockSpec((NB, R * HID), lambda i: (i, 0)),
            pl.BlockSpec((NB, R * HID), lambda i: (i, 0)),
        ],
        out_shape=[_f32(N, HID), _f32(N, R * HID), _f32(N, R * HID)],
    )(x, oh, kW, kb, qW, qb, vW, vb, BDk, BDv)


def _rk2_body(emb, rteW, rteb, kW, vW, BDk, BDv, ko, vo):
    r_vec = jnp.dot(emb[:], rteW[:],
                    preferred_element_type=jnp.float32) + rteb[:]
    for t in range(T):
        RKt = jnp.dot(r_vec, kW[t], preferred_element_type=jnp.float32)
        RVt = jnp.dot(r_vec, vW[t], preferred_element_type=jnp.float32)
        for r in range(R):
            c = (t * R + r) * HID
            ko[:, c:c + HID] = jnp.dot(RKt, BDk[r],
                                       preferred_element_type=jnp.float32)
            vo[:, c:c + HID] = jnp.dot(RVt, BDv[r],
                                       preferred_element_type=jnp.float32)


def _rk2(emb, rteW, rteb, kW, vW, BDk, BDv):
    return pl.pallas_call(
        _rk2_body,
        out_shape=[_f32(ML, T * R * HID), _f32(ML, T * R * HID)],
    )(emb, rteW, rteb.reshape(1, HID), kW, vW, BDk, BDv)


def _merge_body(p, o):
    o[:] = jnp.max(p[:], axis=0)


def _merge(mpart):
    return pl.pallas_call(
        _merge_body,
        out_shape=_f32(N * H),
    )(mpart)


def _combine_body(final, aggp, sp, x, oh, aW, ab, skp, REP, o):
    agg = aggp[0] + aggp[1]
    s16 = sp[0] + sp[1]
    denom = jnp.dot(s16, REP[:], preferred_element_type=jnp.float32) + 1e-16
    aggr = agg / denom
    aggr = 0.5 * aggr * (1.0 + lax.erf(aggr / math.sqrt(2.0)))
    trans = _ptl_block(aggr, oh[:], aW, ab)
    alphas = jax.nn.sigmoid(skp[:])
    alpha = jnp.sum(oh[:] * alphas, axis=1, keepdims=True)
    y = trans * alpha + x[:] * (1.0 - alpha)
    if final:
        y = y / jnp.sqrt(jnp.sum(y * y, axis=-1, keepdims=True))
    o[:] = y


def _combine(aggp, sp, x, oh, aW, ab, skp, final):
    return pl.pallas_call(
        functools.partial(_combine_body, final),
        grid=(GRID,),
        in_specs=[
            pl.BlockSpec((NC, NB, HID), lambda i: (0, i, 0)),
            pl.BlockSpec((NC, NB, 16), lambda i: (0, i, 0)),
            pl.BlockSpec((NB, HID), lambda i: (i, 0)),
            pl.BlockSpec((NB, T), lambda i: (i, 0)),
            pl.BlockSpec((T, HID, HID), lambda i: (0, 0, 0)),
            pl.BlockSpec((T, HID), lambda i: (0, 0)),
            pl.BlockSpec((1, T), lambda i: (0, 0)),
            pl.BlockSpec((16, HID), lambda i: (0, 0)),
        ],
        out_specs=pl.BlockSpec((NB, HID), lambda i: (i, 0)),
        out_shape=_f32(N, HID),
    )(aggp, sp, x, oh, aW, ab, skp.reshape(1, T), _rep_matrix())


def _rep_matrix():
    i = jnp.arange(16)[:, None]
    j = jnp.arange(HID)[None, :]
    return jnp.where((j // DK) == i, 1.0, 0.0).astype(jnp.float32)


def _block_diag(A):
    r = A.shape[0]
    out = jnp.zeros((r, H, DK, H, DK), A.dtype)
    idx = jnp.arange(H)
    out = out.at[:, idx, :, idx, :].set(jnp.moveaxis(A, 1, 0))
    return out.reshape(r, HID, HID)


_SC_PARAMS = pltpu.CompilerParams(
    needs_layout_passes=False, use_tc_tiling_on_sc=False)


@functools.cache
def _mesh():
    return plsc.VectorSubcoreMesh(core_axis_name="c", subcore_axis_name="s",
                                  num_cores=NC, num_subcores=NS)


def _worker_id():
    return lax.axis_index("c") * NS + lax.axis_index("s")


def _nj(nchunk):
    w = _worker_id()
    extra = nchunk - (nchunk // NW) * NW
    return w, jnp.where(w < extra, nchunk // NW + 1, nchunk // NW)


def _iota16():
    return lax.iota(jnp.int32, 16)


def _pipeline(nchunk, load_issue, wait_compute):
    w, nj = _nj(nchunk)

    def ci(i):
        return w + NW * i

    load_issue(ci(0), 0)

    def pair(j, carry):
        i1 = 2 * j + 1
        i2 = 2 * j + 2

        @pl.when(i1 < nj)
        def _():
            load_issue(ci(i1), 1)
        wait_compute(ci(2 * j), 0)

        @pl.when(i2 < nj)
        def _():
            load_issue(ci(i2), 0)

        @pl.when(i1 < nj)
        def _():
            wait_compute(ci(i1), 1)
        return carry

    lax.fori_loop(0, (nj + 1) // 2, pair, 0)


def _pass1a_body(first, *refs):
    if first:
        (qn, kall, rk2, epack, ntr,
         att_o, eout_o,
         ntv, eb0, eb1, ob0, ob1,
         q0, k0, rk0, q1, k1, rk1, attT,
         sq0, sk0, sr0, sq1, sk1, sr1) = refs
        ebufs = (eb0, eb1)
        obufs = (ob0, ob1)
        pltpu.sync_copy(ntr, ntv.at[pl.ds(0, N)])
    else:
        (qn, kall, rk2, epack,
         att_o,
         eb0, eb1,
         q0, k0, rk0, q1, k1, rk1, attT,
         sq0, sk0, sr0, sq1, sk1, sr1) = refs
        ebufs = (eb0, eb1)
        obufs = ebufs
    qb = (q0, q1)
    kb = (k0, k1)
    rkb = (rk0, rk1)
    sems = ((sq0, sk0, sr0), (sq1, sk1, sr1))

    def load_issue(c, b):
        base = c * CB
        eb = ebufs[b]
        ob = obufs[b]
        pltpu.sync_copy(epack.at[:, pl.ds(base, CB)], eb)
        if first:
            for g in range(CB // 16):
                sl = pl.ds(g * 16, 16)
                s16 = eb[0, sl]
                d16 = eb[1, sl]
                e16 = eb[2, sl]
                r16 = eb[3, sl]
                tj = plsc.load_gather(ntv, [s16])
                ob[0, sl] = d16
                ob[1, sl] = s16 * R + r16
                ob[2, sl] = (e16 * T + tj) * R + r16
                ob[3, sl] = d16
            pltpu.sync_copy(ob, eout_o.at[:, pl.ds(base, CB)])
        pltpu.async_copy(qn.at[ob.at[0]], qb[b], sems[b][0])
        pltpu.async_copy(kall.at[ob.at[1]], kb[b], sems[b][1])
        pltpu.async_copy(rk2.at[ob.at[2]], rkb[b], sems[b][2])

    def wait_compute(c, b):
        base = c * CB
        ob = obufs[b]
        pltpu.make_async_copy(qn.at[ob.at[0]], qb[b], sems[b][0]).wait()
        pltpu.make_async_copy(kall.at[ob.at[1]], kb[b], sems[b][1]).wait()
        pltpu.make_async_copy(rk2.at[ob.at[2]], rkb[b], sems[b][2]).wait()

        lane15 = _iota16() == 15

        def edge(e, carry):
            e16 = jnp.full((16,), e, jnp.int32)
            for h in range(H):
                sl = pl.ds(h * DK, DK)
                prod = qb[b][e, sl] * (kb[b][e, sl] + rkb[b][e, sl])
                tot = plsc.cumsum(prod)
                plsc.store_scatter(attT,
                                   [jnp.full((16,), h, jnp.int32), e16],
                                   tot, mask=lane15)
            return carry
        lax.fori_loop(0, CB, edge, 0)
        pltpu.sync_copy(attT, att_o.at[:, pl.ds(base, CB)])

    _pipeline(E // CB, load_issue, wait_compute)


def _row_bufs(cb):
    return [pltpu.VMEM((cb, HID), jnp.float32)] * 3


def _pass1a_first(Qn, Kall, RK2, epack, nt):
    scratch = (
        [pltpu.VMEM((10112,), jnp.int32)]
        + [pltpu.VMEM((4, CB), jnp.int32)] * 4
        + _row_bufs(CB) + _row_bufs(CB)
        + [pltpu.VMEM((H, CB), jnp.float32)]
        + [pltpu.SemaphoreType.DMA] * 6
    )
    fn = pl.kernel(
        functools.partial(_pass1a_body, True),
        out_type=(_f32(H, E), _i32(4, E)),
        mesh=_mesh(),
        scratch_types=scratch,
        compiler_params=_SC_PARAMS,
    )
    return fn(Qn, Kall, RK2, epack, nt)


def _pass1a_rest(Qn, Kall, RK2, eout):
    scratch = (
        [pltpu.VMEM((4, CB), jnp.int32)] * 2
        + _row_bufs(CB) + _row_bufs(CB)
        + [pltpu.VMEM((H, CB), jnp.float32)]
        + [pltpu.SemaphoreType.DMA] * 6
    )
    fn = pl.kernel(
        functools.partial(_pass1a_body, False),
        out_type=_f32(H, E),
        mesh=_mesh(),
        scratch_types=scratch,
        compiler_params=_SC_PARAMS,
    )
    return fn(Qn, Kall, RK2, eout)


def _pass1b_body(attr, eout, mpart_o, dstv, attT, mloc):
    neg = jnp.full((16,), NEG, jnp.float32)

    def init(i, c):
        mloc[pl.ds(i * 16, 16)] = neg
        return c
    lax.fori_loop(0, (N * H) // 16, init, 0)

    w, nj = _nj(E // CB1B)

    def chunk(i, carry):
        base = (w + NW * i) * CB1B
        pltpu.sync_copy(eout.at[0, pl.ds(base, CB1B)], dstv)
        pltpu.sync_copy(attr.at[:, pl.ds(base, CB1B)], attT)

        def grp(g, carry2):
            dst16 = dstv[pl.ds(g * 16, 16)]
            for h in range(H):
                idx = dst16 + h * N
                val = attT[h, pl.ds(g * 16, 16)]
                cur = plsc.load_gather(mloc, [idx])
                msk = val > cur

                def cond(mm):
                    return jnp.any(mm)

                def body(mm):
                    plsc.store_scatter(mloc, [idx], val, mask=mm)
                    cur2 = plsc.load_gather(mloc, [idx])
                    return mm & (val > cur2)
                lax.while_loop(cond, body, msk)
            return carry2
        lax.fori_loop(0, CB1B // 16, grp, 0)
        return carry
    lax.fori_loop(0, nj, chunk, 0)
    pltpu.sync_copy(mloc, mpart_o.at[_worker_id()])


def _pass1b(att, eout):
    scratch = [
        pltpu.VMEM((CB1B,), jnp.int32),
        pltpu.VMEM((H, CB1B), jnp.float32),
        pltpu.VMEM((N * H,), jnp.float32),
    ]
    fn = pl.kernel(
        _pass1b_body,
        out_type=_f32(NW, N * H),
        mesh=_mesh(),
        scratch_types=scratch,
        compiler_params=_SC_PARAMS,
    )
    return fn(att, eout)


ROWS_PER_TILE = N // NS
ZCH = 5


def _pass2_body(attr, mr, eout, vall, rv2,
                aggp_o, sp_o,
                eb0, eb1, m0, m1, v0, rv0, v1, rv1, a0, a1,
                msgrows, wT, wrows, zbuf, zs, agg_s, s_s,
                sm0, sv0, sr0, sm1, sv1, sr1):
    cid = lax.axis_index("c")
    sid = lax.axis_index("s")
    r0 = sid * ROWS_PER_TILE
    ebufs = (eb0, eb1)
    mb = (m0, m1)
    vb = (v0, v1)
    rvb = (rv0, rv1)
    ab = (a0, a1)
    sems = ((sm0, sv0, sr0), (sm1, sv1, sr1))

    z16 = jnp.zeros((16,), jnp.float32)

    def zinit(i, c):
        for cc in range(HID // 16):
            zbuf[i, pl.ds(cc * 16, 16)] = z16
        zs[i, :] = z16
        return c
    lax.fori_loop(0, ZCH, zinit, 0)

    def winit(i, c):
        wrows[i, :] = z16
        return c
    lax.fori_loop(0, CB2, winit, 0)

    for j in range(ROWS_PER_TILE // ZCH):
        pltpu.sync_copy(zbuf, agg_s.at[pl.ds(r0 + j * ZCH, ZCH)])
        pltpu.sync_copy(zs, s_s.at[pl.ds(r0 + j * ZCH, ZCH)])
    plsc.subcore_barrier()

    def load_issue(c, b):
        base = c * CB2
        eb = ebufs[b]
        pltpu.sync_copy(eout.at[:, pl.ds(base, CB2)], eb)
        pltpu.sync_copy(attr.at[:, pl.ds(base, CB2)], ab[b])
        pltpu.async_copy(mr.at[eb.at[0]], mb[b], sems[b][0])
        pltpu.async_copy(vall.at[eb.at[1]], vb[b], sems[b][1])
        pltpu.async_copy(rv2.at[eb.at[2]], rvb[b], sems[b][2])

    def wait_compute(c, b):
        eb = ebufs[b]
        pltpu.make_async_copy(mr.at[eb.at[0]], mb[b], sems[b][0]).wait()
        pltpu.make_async_copy(vall.at[eb.at[1]], vb[b], sems[b][1]).wait()
        pltpu.make_async_copy(rv2.at[eb.at[2]], rvb[b], sems[b][2]).wait()

        for g in range(CB2 // 16):
            erow = g * 16 + _iota16()
            for h in range(H):
                hcol = jnp.full((16,), h, jnp.int32)
                m16 = plsc.load_gather(mb[b], [erow, hcol])
                a16 = ab[b][h, pl.ds(g * 16, 16)]
                w16 = jnp.exp(a16 - m16)
                wT[h, pl.ds(g * 16, 16)] = w16
                plsc.store_scatter(wrows, [erow, hcol], w16)

        def edge(e, carry):
            e16 = jnp.full((16,), e, jnp.int32)
            for h in range(H):
                wb = plsc.load_gather(wT, [jnp.full((16,), h, jnp.int32), e16])
                v16 = vb[b][e, pl.ds(h * DK, DK)]
                rv16 = rvb[b][e, pl.ds(h * DK, DK)]
                msgrows[e, pl.ds(h * DK, DK)] = wb * (v16 + rv16)
            return carry
        lax.fori_loop(0, CB2, edge, 0)

        pltpu.sync_copy(msgrows, agg_s.at[eb.at[0]], add=True)
        pltpu.sync_copy(wrows, s_s.at[eb.at[0]], add=True)

    _pipeline(E // CB2, load_issue, wait_compute)
    plsc.subcore_barrier()

    for j in range(ROWS_PER_TILE // ZCH):
        pltpu.sync_copy(agg_s.at[pl.ds(r0 + j * ZCH, ZCH)], zbuf)
        pltpu.sync_copy(zbuf, aggp_o.at[cid, pl.ds(r0 + j * ZCH, ZCH)])
        pltpu.sync_copy(s_s.at[pl.ds(r0 + j * ZCH, ZCH)], zs)
        pltpu.sync_copy(zs, sp_o.at[cid, pl.ds(r0 + j * ZCH, ZCH)])


def _pass2(att, m2, eout, Vall, RV2):
    scratch = [
        pltpu.VMEM((4, CB2), jnp.int32),
        pltpu.VMEM((4, CB2), jnp.int32),
        pltpu.VMEM((CB2, 16), jnp.float32),
        pltpu.VMEM((CB2, 16), jnp.float32),
        pltpu.VMEM((CB2, HID), jnp.float32),
        pltpu.VMEM((CB2, HID), jnp.float32),
        pltpu.VMEM((CB2, HID), jnp.float32),
        pltpu.VMEM((CB2, HID), jnp.float32),
        pltpu.VMEM((H, CB2), jnp.float32),
        pltpu.VMEM((H, CB2), jnp.float32),
        pltpu.VMEM((CB2, HID), jnp.float32),
        pltpu.VMEM((H, CB2), jnp.float32),
        pltpu.VMEM((CB2, 16), jnp.float32),
        pltpu.VMEM((ZCH, HID), jnp.float32),
        pltpu.VMEM((ZCH, 16), jnp.float32),
        pltpu.VMEM_SHARED((N, HID), jnp.float32),
        pltpu.VMEM_SHARED((N, 16), jnp.float32),
    ] + [pltpu.SemaphoreType.DMA] * 6
    fn = pl.kernel(
        _pass2_body,
        out_type=(_f32(NC, N, HID), _f32(NC, N, 16)),
        mesh=_mesh(),
        scratch_types=scratch,
        compiler_params=_SC_PARAMS,
    )
    return fn(att, m2, eout, Vall, RV2)


def kernel(node_feature, adapt_W, adapt_b, k_W, k_b, q_W, q_b, v_W, v_b,
           a_W, a_b, rel_pri, rel_att, rel_msg, skip, rte_W, rte_b,
           rte_emb, node_type, edge_time, edge_index, edge_type):
    nt = node_type.astype(jnp.int32)
    epack = jnp.stack([edge_index[0].astype(jnp.int32),
                       edge_index[1].astype(jnp.int32),
                       edge_time.astype(jnp.int32),
                       edge_type.astype(jnp.int32)])
    oh = jax.nn.one_hot(nt, T, dtype=jnp.float32)

    x = _adapt(node_feature, oh, adapt_W, adapt_b)
    eout = None
    for l in range(L):
        scale = rel_pri[l] / SQRT_DK
        BDk = _block_diag(rel_att[l] * scale[..., None, None])
        BDv = _block_diag(rel_msg[l])
        Qn, Kall, Vall = _proj(x, oh, k_W[l], k_b[l], q_W[l], q_b[l],
                               v_W[l], v_b[l], BDk, BDv)
        RK2, RV2 = _rk2(rte_emb, rte_W[l], rte_b[l], k_W[l], v_W[l], BDk, BDv)
        Kall = Kall.reshape(N * R, HID)
        Vall = Vall.reshape(N * R, HID)
        RK2 = RK2.reshape(ML * T * R, HID)
        RV2 = RV2.reshape(ML * T * R, HID)
        if l == 0:
            att, eout = _pass1a_first(Qn, Kall, RK2, epack, nt)
        else:
            att = _pass1a_rest(Qn, Kall, RK2, eout)
        mpart = _pass1b(att, eout)
        m = _merge(mpart)
        m2 = jnp.pad(m.reshape(H, N).T, ((0, 0), (0, 16 - H)))
        aggp, sp = _pass2(att, m2, eout, Vall, RV2)
        x = _combine(aggp, sp, x, oh, a_W[l], a_b[l], skip[l],
                     final=(l == L - 1))
    return x


# speedup vs baseline: 6.7890x; 1.1440x over previous
import functools
import math

import jax
import jax.numpy as jnp
from jax import lax
from jax.experimental import pallas as pl
from jax.experimental.pallas import tpu as pltpu
from jax.experimental.pallas import tpu_sc as plsc

N = 10000
E = 320000
IN_DIM = 128
HID = 128
T = 4
R = 8
H = 8
DK = HID // H
L = 2
ML = 240
SQRT_DK = math.sqrt(DK)

NC = 2
NS = 16
NW = NC * NS
CB = 128
CB2 = 32
CB1B = 512
NB = 1000
GRID = N // NB
NEG = -3.0e38


def _f32(*shape):
    return jax.ShapeDtypeStruct(shape, jnp.float32)


def _i32(*shape):
    return jax.ShapeDtypeStruct(shape, jnp.int32)


def _adapt_body(nf, oh, W, b, o):
    x = nf[:]
    acc = jnp.zeros((NB, HID), jnp.float32)
    for t in range(T):
        y = jnp.dot(x, W[t], preferred_element_type=jnp.float32) + b[t][None, :]
        acc = acc + oh[:, t][:, None] * y
    o[:] = jnp.tanh(acc)


def _adapt(nf, oh, W, b):
    return pl.pallas_call(
        _adapt_body,
        grid=(GRID,),
        in_specs=[
            pl.BlockSpec((NB, IN_DIM), lambda i: (i, 0)),
            pl.BlockSpec((NB, T), lambda i: (i, 0)),
            pl.BlockSpec((T, IN_DIM, HID), lambda i: (0, 0, 0)),
            pl.BlockSpec((T, HID), lambda i: (0, 0)),
        ],
        out_specs=pl.BlockSpec((NB, HID), lambda i: (i, 0)),
        out_shape=_f32(N, HID),
    )(nf, oh, W, b)


def _ptl_block(x, oh, W, b):
    acc = jnp.zeros((x.shape[0], HID), jnp.float32)
    for t in range(T):
        y = jnp.dot(x, W[t], preferred_element_type=jnp.float32) + b[t][None, :]
        acc = acc + oh[:, t][:, None] * y
    return acc


def _proj_body(x, oh, kW, kb, qW, qb, vW, vb, BDk, BDv, qo, ko, vo):
    xb = x[:]
    ohb = oh[:]
    Kn = _ptl_block(xb, ohb, kW, kb)
    qo[:] = _ptl_block(xb, ohb, qW, qb)
    Vn = _ptl_block(xb, ohb, vW, vb)
    for r in range(R):
        ko[:, r * HID:(r + 1) * HID] = jnp.dot(
            Kn, BDk[r], preferred_element_type=jnp.float32)
        vo[:, r * HID:(r + 1) * HID] = jnp.dot(
            Vn, BDv[r], preferred_element_type=jnp.float32)


def _proj(x, oh, kW, kb, qW, qb, vW, vb, BDk, BDv):
    wspec3 = pl.BlockSpec((T, HID, HID), lambda i: (0, 0, 0))
    wspec2 = pl.BlockSpec((T, HID), lambda i: (0, 0))
    bdspec = pl.BlockSpec((R, HID, HID), lambda i: (0, 0, 0))
    return pl.pallas_call(
        _proj_body,
        grid=(GRID,),
        in_specs=[
            pl.BlockSpec((NB, HID), lambda i: (i, 0)),
            pl.BlockSpec((NB, T), lambda i: (i, 0)),
            wspec3, wspec2, wspec3, wspec2, wspec3, wspec2, bdspec, bdspec,
        ],
        out_specs=[
            pl.BlockSpec((NB, HID), lambda i: (i, 0)),
            pl.BlockSpec((NB, R * HID), lambda i: (i, 0)),
            pl.BlockSpec((NB, R * HID), lambda i: (i, 0)),
        ],
        out_shape=[_f32(N, HID), _f32(N, R * HID), _f32(N, R * HID)],
    )(x, oh, kW, kb, qW, qb, vW, vb, BDk, BDv)


def _rk2_body(emb, rteW, rteb, kW, vW, BDk, BDv, ko, vo):
    r_vec = jnp.dot(emb[:], rteW[:],
                    preferred_element_type=jnp.float32) + rteb[:]
    for t in range(T):
        RKt = jnp.dot(r_vec, kW[t], preferred_element_type=jnp.float32)
        RVt = jnp.dot(r_vec, vW[t], preferred_element_type=jnp.float32)
        for r in range(R):
            c = (t * R + r) * HID
            ko[:, c:c + HID] = jnp.dot(RKt, BDk[r],
                                       preferred_element_type=jnp.float32)
            vo[:, c:c + HID] = jnp.dot(RVt, BDv[r],
                                       preferred_element_type=jnp.float32)


def _rk2(emb, rteW, rteb, kW, vW, BDk, BDv):
    return pl.pallas_call(
        _rk2_body,
        out_shape=[_f32(ML, T * R * HID), _f32(ML, T * R * HID)],
    )(emb, rteW, rteb.reshape(1, HID), kW, vW, BDk, BDv)


def _merge_body(p, o):
    o[:] = jnp.max(p[:], axis=0)


def _merge(mpart):
    return pl.pallas_call(
        _merge_body,
        out_shape=_f32(N * H),
    )(mpart)


def _combine_body(final, aggp, sp, x, oh, aW, ab, skp, REP, o):
    agg = aggp[0] + aggp[1]
    s16 = sp[0] + sp[1]
    denom = jnp.dot(s16, REP[:], preferred_element_type=jnp.float32) + 1e-16
    aggr = agg / denom
    aggr = 0.5 * aggr * (1.0 + lax.erf(aggr / math.sqrt(2.0)))
    trans = _ptl_block(aggr, oh[:], aW, ab)
    alphas = jax.nn.sigmoid(skp[:])
    alpha = jnp.sum(oh[:] * alphas, axis=1, keepdims=True)
    y = trans * alpha + x[:] * (1.0 - alpha)
    if final:
        y = y / jnp.sqrt(jnp.sum(y * y, axis=-1, keepdims=True))
    o[:] = y


def _combine(aggp, sp, x, oh, aW, ab, skp, final):
    return pl.pallas_call(
        functools.partial(_combine_body, final),
        grid=(GRID,),
        in_specs=[
            pl.BlockSpec((NC, NB, HID), lambda i: (0, i, 0)),
            pl.BlockSpec((NC, NB, 16), lambda i: (0, i, 0)),
            pl.BlockSpec((NB, HID), lambda i: (i, 0)),
            pl.BlockSpec((NB, T), lambda i: (i, 0)),
            pl.BlockSpec((T, HID, HID), lambda i: (0, 0, 0)),
            pl.BlockSpec((T, HID), lambda i: (0, 0)),
            pl.BlockSpec((1, T), lambda i: (0, 0)),
            pl.BlockSpec((16, HID), lambda i: (0, 0)),
        ],
        out_specs=pl.BlockSpec((NB, HID), lambda i: (i, 0)),
        out_shape=_f32(N, HID),
    )(aggp, sp, x, oh, aW, ab, skp.reshape(1, T), _rep_matrix())


def _rep_matrix():
    i = jnp.arange(16)[:, None]
    j = jnp.arange(HID)[None, :]
    return jnp.where((j // DK) == i, 1.0, 0.0).astype(jnp.float32)


def _block_diag(A):
    r = A.shape[0]
    out = jnp.zeros((r, H, DK, H, DK), A.dtype)
    idx = jnp.arange(H)
    out = out.at[:, idx, :, idx, :].set(jnp.moveaxis(A, 1, 0))
    return out.reshape(r, HID, HID)


_SC_PARAMS = pltpu.CompilerParams(
    needs_layout_passes=False, use_tc_tiling_on_sc=False)


@functools.cache
def _mesh():
    return plsc.VectorSubcoreMesh(core_axis_name="c", subcore_axis_name="s",
                                  num_cores=NC, num_subcores=NS)


def _worker_id():
    return lax.axis_index("c") * NS + lax.axis_index("s")


def _nj(nchunk):
    w = _worker_id()
    extra = nchunk - (nchunk // NW) * NW
    return w, jnp.where(w < extra, nchunk // NW + 1, nchunk // NW)


def _iota16():
    return lax.iota(jnp.int32, 16)


def _pipeline(nchunk, load_issue, wait_compute):
    w, nj = _nj(nchunk)

    def ci(i):
        return w + NW * i

    load_issue(ci(0), 0)

    def pair(j, carry):
        i1 = 2 * j + 1
        i2 = 2 * j + 2

        @pl.when(i1 < nj)
        def _():
            load_issue(ci(i1), 1)
        wait_compute(ci(2 * j), 0)

        @pl.when(i2 < nj)
        def _():
            load_issue(ci(i2), 0)

        @pl.when(i1 < nj)
        def _():
            wait_compute(ci(i1), 1)
        return carry

    lax.fori_loop(0, (nj + 1) // 2, pair, 0)


def _pass1a_body(first, *refs):
    if first:
        (qn, kall, rk2, epack, ntr,
         att_o, eout_o,
         ntv, eb0, eb1, ob0, ob1,
         q0, k0, rk0, q1, k1, rk1, attT,
         sq0, sk0, sr0, sq1, sk1, sr1) = refs
        ebufs = (eb0, eb1)
        obufs = (ob0, ob1)
        pltpu.sync_copy(ntr, ntv.at[pl.ds(0, N)])
    else:
        (qn, kall, rk2, epack,
         att_o,
         eb0, eb1,
         q0, k0, rk0, q1, k1, rk1, attT,
         sq0, sk0, sr0, sq1, sk1, sr1) = refs
        ebufs = (eb0, eb1)
        obufs = ebufs
    qb = (q0, q1)
    kb = (k0, k1)
    rkb = (rk0, rk1)
    sems = ((sq0, sk0, sr0), (sq1, sk1, sr1))

    def load_issue(c, b):
        base = c * CB
        eb = ebufs[b]
        ob = obufs[b]
        pltpu.sync_copy(epack.at[:, pl.ds(base, CB)], eb)
        if first:
            for g in range(CB // 16):
                sl = pl.ds(g * 16, 16)
                s16 = eb[0, sl]
                d16 = eb[1, sl]
                e16 = eb[2, sl]
                r16 = eb[3, sl]
                tj = plsc.load_gather(ntv, [s16])
                ob[0, sl] = d16
                ob[1, sl] = s16 * R + r16
                ob[2, sl] = (e16 * T + tj) * R + r16
                ob[3, sl] = d16
            pltpu.sync_copy(ob, eout_o.at[:, pl.ds(base, CB)])
        pltpu.async_copy(qn.at[ob.at[0]], qb[b], sems[b][0])
        pltpu.async_copy(kall.at[ob.at[1]], kb[b], sems[b][1])
        pltpu.async_copy(rk2.at[ob.at[2]], rkb[b], sems[b][2])

    def wait_compute(c, b):
        base = c * CB
        ob = obufs[b]
        pltpu.make_async_copy(qn.at[ob.at[0]], qb[b], sems[b][0]).wait()
        pltpu.make_async_copy(kall.at[ob.at[1]], kb[b], sems[b][1]).wait()
        pltpu.make_async_copy(rk2.at[ob.at[2]], rkb[b], sems[b][2]).wait()

        lane15 = _iota16() == 15

        def edge(e, carry):
            e16 = jnp.full((16,), e, jnp.int32)
            for h in range(H):
                sl = pl.ds(h * DK, DK)
                prod = qb[b][e, sl] * (kb[b][e, sl] + rkb[b][e, sl])
                tot = plsc.cumsum(prod)
                plsc.store_scatter(attT,
                                   [jnp.full((16,), h, jnp.int32), e16],
                                   tot, mask=lane15)
            return carry
        lax.fori_loop(0, CB, edge, 0)
        pltpu.sync_copy(attT, att_o.at[:, pl.ds(base, CB)])

    _pipeline(E // CB, load_issue, wait_compute)


def _row_bufs(cb):
    return [pltpu.VMEM((cb, HID), jnp.float32)] * 3


def _pass1a_first(Qn, Kall, RK2, epack, nt):
    scratch = (
        [pltpu.VMEM((10112,), jnp.int32)]
        + [pltpu.VMEM((4, CB), jnp.int32)] * 4
        + _row_bufs(CB) + _row_bufs(CB)
        + [pltpu.VMEM((H, CB), jnp.float32)]
        + [pltpu.SemaphoreType.DMA] * 6
    )
    fn = pl.kernel(
        functools.partial(_pass1a_body, True),
        out_type=(_f32(H, E), _i32(4, E)),
        mesh=_mesh(),
        scratch_types=scratch,
        compiler_params=_SC_PARAMS,
    )
    return fn(Qn, Kall, RK2, epack, nt)


def _pass1a_rest(Qn, Kall, RK2, eout):
    scratch = (
        [pltpu.VMEM((4, CB), jnp.int32)] * 2
        + _row_bufs(CB) + _row_bufs(CB)
        + [pltpu.VMEM((H, CB), jnp.float32)]
        + [pltpu.SemaphoreType.DMA] * 6
    )
    fn = pl.kernel(
        functools.partial(_pass1a_body, False),
        out_type=_f32(H, E),
        mesh=_mesh(),
        scratch_types=scratch,
        compiler_params=_SC_PARAMS,
    )
    return fn(Qn, Kall, RK2, eout)


def _pass1b_body(attr, eout, mpart_o, dstv, attT, mloc):
    neg = jnp.full((16,), NEG, jnp.float32)

    def init(i, c):
        mloc[pl.ds(i * 16, 16)] = neg
        return c
    lax.fori_loop(0, (N * H) // 16, init, 0)

    w, nj = _nj(E // CB1B)

    def chunk(i, carry):
        base = (w + NW * i) * CB1B
        pltpu.sync_copy(eout.at[0, pl.ds(base, CB1B)], dstv)
        pltpu.sync_copy(attr.at[:, pl.ds(base, CB1B)], attT)

        def grp(g, carry2):
            dst16 = dstv[pl.ds(g * 16, 16)]
            for h in range(H):
                idx = dst16 + h * N
                val = attT[h, pl.ds(g * 16, 16)]
                cur = plsc.load_gather(mloc, [idx])
                msk = val > cur

                def cond(mm):
                    return jnp.any(mm)

                def body(mm):
                    plsc.store_scatter(mloc, [idx], val, mask=mm)
                    cur2 = plsc.load_gather(mloc, [idx])
                    return mm & (val > cur2)
                lax.while_loop(cond, body, msk)
            return carry2
        lax.fori_loop(0, CB1B // 16, grp, 0)
        return carry
    lax.fori_loop(0, nj, chunk, 0)
    pltpu.sync_copy(mloc, mpart_o.at[_worker_id()])


def _pass1b(att, eout):
    scratch = [
        pltpu.VMEM((CB1B,), jnp.int32),
        pltpu.VMEM((H, CB1B), jnp.float32),
        pltpu.VMEM((N * H,), jnp.float32),
    ]
    fn = pl.kernel(
        _pass1b_body,
        out_type=_f32(NW, N * H),
        mesh=_mesh(),
        scratch_types=scratch,
        compiler_params=_SC_PARAMS,
    )
    return fn(att, eout)


ROWS_PER_TILE = N // NS
ZCH = 5


def _pass2_body(attr, mr, eout, vall, rv2,
                aggp_o, sp_o,
                eb0, eb1, m0, m1, v0, rv0, v1, rv1, a0, a1,
                msgrows, wT, wrows, zbuf, zs, agg_s, s_s,
                sm0, sv0, sr0, sm1, sv1, sr1):
    cid = lax.axis_index("c")
    sid = lax.axis_index("s")
    r0 = sid * ROWS_PER_TILE
    ebufs = (eb0, eb1)
    mb = (m0, m1)
    vb = (v0, v1)
    rvb = (rv0, rv1)
    ab = (a0, a1)
    sems = ((sm0, sv0, sr0), (sm1, sv1, sr1))

    z16 = jnp.zeros((16,), jnp.float32)

    def zinit(i, c):
        for cc in range(HID // 16):
            zbuf[i, pl.ds(cc * 16, 16)] = z16
        zs[i, :] = z16
        return c
    lax.fori_loop(0, ZCH, zinit, 0)

    def winit(i, c):
        wrows[i, :] = z16
        return c
    lax.fori_loop(0, CB2, winit, 0)

    for j in range(ROWS_PER_TILE // ZCH):
        pltpu.sync_copy(zbuf, agg_s.at[pl.ds(r0 + j * ZCH, ZCH)])
        pltpu.sync_copy(zs, s_s.at[pl.ds(r0 + j * ZCH, ZCH)])
    plsc.subcore_barrier()

    def load_issue(c, b):
        base = c * CB2
        eb = ebufs[b]
        pltpu.sync_copy(eout.at[:, pl.ds(base, CB2)], eb)
        pltpu.sync_copy(attr.at[:, pl.ds(base, CB2)], ab[b])
        pltpu.async_copy(mr.at[eb.at[0]], mb[b], sems[b][0])
        pltpu.async_copy(vall.at[eb.at[1]], vb[b], sems[b][1])
        pltpu.async_copy(rv2.at[eb.at[2]], rvb[b], sems[b][2])

    def wait_compute(c, b):
        eb = ebufs[b]
        pltpu.make_async_copy(mr.at[eb.at[0]], mb[b], sems[b][0]).wait()
        pltpu.make_async_copy(vall.at[eb.at[1]], vb[b], sems[b][1]).wait()
        pltpu.make_async_copy(rv2.at[eb.at[2]], rvb[b], sems[b][2]).wait()

        for g in range(CB2 // 16):
            erow = g * 16 + _iota16()
            for h in range(H):
                hcol = jnp.full((16,), h, jnp.int32)
                m16 = plsc.load_gather(mb[b], [erow, hcol])
                a16 = ab[b][h, pl.ds(g * 16, 16)]
                w16 = jnp.exp(a16 - m16)
                wT[h, pl.ds(g * 16, 16)] = w16
                plsc.store_scatter(wrows, [erow, hcol], w16)

        iot = _iota16()

        def edge(e, carry):
            wrow = wrows[e, :]
            for h in range(H):
                c1 = plsc.cumsum(jnp.where(iot == h, wrow, 0.0))
                wb = jnp.maximum(c1, lax.rev(c1, (0,)))
                v16 = vb[b][e, pl.ds(h * DK, DK)]
                rv16 = rvb[b][e, pl.ds(h * DK, DK)]
                msgrows[e, pl.ds(h * DK, DK)] = wb * (v16 + rv16)
            return carry
        lax.fori_loop(0, CB2, edge, 0)

        pltpu.sync_copy(msgrows, agg_s.at[eb.at[0]], add=True)
        pltpu.sync_copy(wrows, s_s.at[eb.at[0]], add=True)

    _pipeline(E // CB2, load_issue, wait_compute)
    plsc.subcore_barrier()

    for j in range(ROWS_PER_TILE // ZCH):
        pltpu.sync_copy(agg_s.at[pl.ds(r0 + j * ZCH, ZCH)], zbuf)
        pltpu.sync_copy(zbuf, aggp_o.at[cid, pl.ds(r0 + j * ZCH, ZCH)])
        pltpu.sync_copy(s_s.at[pl.ds(r0 + j * ZCH, ZCH)], zs)
        pltpu.sync_copy(zs, sp_o.at[cid, pl.ds(r0 + j * ZCH, ZCH)])


def _pass2(att, m2, eout, Vall, RV2):
    scratch = [
        pltpu.VMEM((4, CB2), jnp.int32),
        pltpu.VMEM((4, CB2), jnp.int32),
        pltpu.VMEM((CB2, 16), jnp.float32),
        pltpu.VMEM((CB2, 16), jnp.float32),
        pltpu.VMEM((CB2, HID), jnp.float32),
        pltpu.VMEM((CB2, HID), jnp.float32),
        pltpu.VMEM((CB2, HID), jnp.float32),
        pltpu.VMEM((CB2, HID), jnp.float32),
        pltpu.VMEM((H, CB2), jnp.float32),
        pltpu.VMEM((H, CB2), jnp.float32),
        pltpu.VMEM((CB2, HID), jnp.float32),
        pltpu.VMEM((H, CB2), jnp.float32),
        pltpu.VMEM((CB2, 16), jnp.float32),
        pltpu.VMEM((ZCH, HID), jnp.float32),
        pltpu.VMEM((ZCH, 16), jnp.float32),
        pltpu.VMEM_SHARED((N, HID), jnp.float32),
        pltpu.VMEM_SHARED((N, 16), jnp.float32),
    ] + [pltpu.SemaphoreType.DMA] * 6
    fn = pl.kernel(
        _pass2_body,
        out_type=(_f32(NC, N, HID), _f32(NC, N, 16)),
        mesh=_mesh(),
        scratch_types=scratch,
        compiler_params=_SC_PARAMS,
    )
    return fn(att, m2, eout, Vall, RV2)


def kernel(node_feature, adapt_W, adapt_b, k_W, k_b, q_W, q_b, v_W, v_b,
           a_W, a_b, rel_pri, rel_att, rel_msg, skip, rte_W, rte_b,
           rte_emb, node_type, edge_time, edge_index, edge_type):
    nt = node_type.astype(jnp.int32)
    epack = jnp.stack([edge_index[0].astype(jnp.int32),
                       edge_index[1].astype(jnp.int32),
                       edge_time.astype(jnp.int32),
                       edge_type.astype(jnp.int32)])
    oh = jax.nn.one_hot(nt, T, dtype=jnp.float32)

    x = _adapt(node_feature, oh, adapt_W, adapt_b)
    eout = None
    for l in range(L):
        scale = rel_pri[l] / SQRT_DK
        BDk = _block_diag(rel_att[l] * scale[..., None, None])
        BDv = _block_diag(rel_msg[l])
        Qn, Kall, Vall = _proj(x, oh, k_W[l], k_b[l], q_W[l], q_b[l],
                               v_W[l], v_b[l], BDk, BDv)
        RK2, RV2 = _rk2(rte_emb, rte_W[l], rte_b[l], k_W[l], v_W[l], BDk, BDv)
        Kall = Kall.reshape(N * R, HID)
        Vall = Vall.reshape(N * R, HID)
        RK2 = RK2.reshape(ML * T * R, HID)
        RV2 = RV2.reshape(ML * T * R, HID)
        if l == 0:
            att, eout = _pass1a_first(Qn, Kall, RK2, epack, nt)
        else:
            att = _pass1a_rest(Qn, Kall, RK2, eout)
        mpart = _pass1b(att, eout)
        m = _merge(mpart)
        m2 = jnp.pad(m.reshape(H, N).T, ((0, 0), (0, 16 - H)))
        aggp, sp = _pass2(att, m2, eout, Vall, RV2)
        x = _combine(aggp, sp, x, oh, a_W[l], a_b[l], skip[l],
                     final=(l == L - 1))
    return x


# speedup vs baseline: 10.6987x; 1.5759x over previous
import functools
import math

import jax
import jax.numpy as jnp
from jax import lax
from jax.experimental import pallas as pl
from jax.experimental.pallas import tpu as pltpu
from jax.experimental.pallas import tpu_sc as plsc

N = 10000
E = 320000
IN_DIM = 128
HID = 128
T = 4
R = 8
H = 8
DK = HID // H
L = 2
ML = 240
SQRT_DK = math.sqrt(DK)

NC = 2
NS = 16
NW = NC * NS
CB = 128
CB2 = 32
CB1B = 512
NB = 1000
GRID = N // NB
NEG = -3.0e38


def _f32(*shape):
    return jax.ShapeDtypeStruct(shape, jnp.float32)


def _i32(*shape):
    return jax.ShapeDtypeStruct(shape, jnp.int32)


def _adapt_body(nf, oh, W, b, o):
    x = nf[:]
    acc = jnp.zeros((NB, HID), jnp.float32)
    for t in range(T):
        y = jnp.dot(x, W[t], preferred_element_type=jnp.float32) + b[t][None, :]
        acc = acc + oh[:, t][:, None] * y
    o[:] = jnp.tanh(acc)


def _adapt(nf, oh, W, b):
    return pl.pallas_call(
        _adapt_body,
        grid=(GRID,),
        in_specs=[
            pl.BlockSpec((NB, IN_DIM), lambda i: (i, 0)),
            pl.BlockSpec((NB, T), lambda i: (i, 0)),
            pl.BlockSpec((T, IN_DIM, HID), lambda i: (0, 0, 0)),
            pl.BlockSpec((T, HID), lambda i: (0, 0)),
        ],
        out_specs=pl.BlockSpec((NB, HID), lambda i: (i, 0)),
        out_shape=_f32(N, HID),
    )(nf, oh, W, b)


def _ptl_block(x, oh, W, b):
    acc = jnp.zeros((x.shape[0], HID), jnp.float32)
    for t in range(T):
        y = jnp.dot(x, W[t], preferred_element_type=jnp.float32) + b[t][None, :]
        acc = acc + oh[:, t][:, None] * y
    return acc


def _proj_body(x, oh, kW, kb, qW, qb, vW, vb, BDk, BDv, qo, ko, vo):
    xb = x[:]
    ohb = oh[:]
    Kn = _ptl_block(xb, ohb, kW, kb)
    qo[:] = _ptl_block(xb, ohb, qW, qb)
    Vn = _ptl_block(xb, ohb, vW, vb)
    for r in range(R):
        ko[:, r * HID:(r + 1) * HID] = jnp.dot(
            Kn, BDk[r], preferred_element_type=jnp.float32)
        vo[:, r * HID:(r + 1) * HID] = jnp.dot(
            Vn, BDv[r], preferred_element_type=jnp.float32)


def _proj(x, oh, kW, kb, qW, qb, vW, vb, BDk, BDv):
    wspec3 = pl.BlockSpec((T, HID, HID), lambda i: (0, 0, 0))
    wspec2 = pl.BlockSpec((T, HID), lambda i: (0, 0))
    bdspec = pl.BlockSpec((R, HID, HID), lambda i: (0, 0, 0))
    return pl.pallas_call(
        _proj_body,
        grid=(GRID,),
        in_specs=[
            pl.BlockSpec((NB, HID), lambda i: (i, 0)),
            pl.BlockSpec((NB, T), lambda i: (i, 0)),
            wspec3, wspec2, wspec3, wspec2, wspec3, wspec2, bdspec, bdspec,
        ],
        out_specs=[
            pl.BlockSpec((NB, HID), lambda i: (i, 0)),
            pl.BlockSpec((NB, R * HID), lambda i: (i, 0)),
            pl.BlockSpec((NB, R * HID), lambda i: (i, 0)),
        ],
        out_shape=[_f32(N, HID), _f32(N, R * HID), _f32(N, R * HID)],
    )(x, oh, kW, kb, qW, qb, vW, vb, BDk, BDv)


def _rk2_body(emb, rteW, rteb, kW, vW, BDk, BDv, ko, vo):
    r_vec = jnp.dot(emb[:], rteW[:],
                    preferred_element_type=jnp.float32) + rteb[:]
    for t in range(T):
        RKt = jnp.dot(r_vec, kW[t], preferred_element_type=jnp.float32)
        RVt = jnp.dot(r_vec, vW[t], preferred_element_type=jnp.float32)
        for r in range(R):
            c = (t * R + r) * HID
            ko[:, c:c + HID] = jnp.dot(RKt, BDk[r],
                                       preferred_element_type=jnp.float32)
            vo[:, c:c + HID] = jnp.dot(RVt, BDv[r],
                                       preferred_element_type=jnp.float32)


def _rk2(emb, rteW, rteb, kW, vW, BDk, BDv):
    return pl.pallas_call(
        _rk2_body,
        out_shape=[_f32(ML, T * R * HID), _f32(ML, T * R * HID)],
    )(emb, rteW, rteb.reshape(1, HID), kW, vW, BDk, BDv)


def _merge_body(p, o):
    o[:] = jnp.max(p[:], axis=0)


def _merge(mpart):
    return pl.pallas_call(
        _merge_body,
        out_shape=_f32(N * H),
    )(mpart)


def _combine_body(final, aggp, sp, x, oh, aW, ab, skp, REP, o):
    agg = aggp[0] + aggp[1]
    s16 = sp[0] + sp[1]
    denom = jnp.dot(s16, REP[:], preferred_element_type=jnp.float32) + 1e-16
    aggr = agg / denom
    aggr = 0.5 * aggr * (1.0 + lax.erf(aggr / math.sqrt(2.0)))
    trans = _ptl_block(aggr, oh[:], aW, ab)
    alphas = jax.nn.sigmoid(skp[:])
    alpha = jnp.sum(oh[:] * alphas, axis=1, keepdims=True)
    y = trans * alpha + x[:] * (1.0 - alpha)
    if final:
        y = y / jnp.sqrt(jnp.sum(y * y, axis=-1, keepdims=True))
    o[:] = y


def _combine(aggp, sp, x, oh, aW, ab, skp, final):
    return pl.pallas_call(
        functools.partial(_combine_body, final),
        grid=(GRID,),
        in_specs=[
            pl.BlockSpec((NC, NB, HID), lambda i: (0, i, 0)),
            pl.BlockSpec((NC, NB, 16), lambda i: (0, i, 0)),
            pl.BlockSpec((NB, HID), lambda i: (i, 0)),
            pl.BlockSpec((NB, T), lambda i: (i, 0)),
            pl.BlockSpec((T, HID, HID), lambda i: (0, 0, 0)),
            pl.BlockSpec((T, HID), lambda i: (0, 0)),
            pl.BlockSpec((1, T), lambda i: (0, 0)),
            pl.BlockSpec((16, HID), lambda i: (0, 0)),
        ],
        out_specs=pl.BlockSpec((NB, HID), lambda i: (i, 0)),
        out_shape=_f32(N, HID),
    )(aggp, sp, x, oh, aW, ab, skp.reshape(1, T), _rep_matrix())


def _rep_matrix():
    i = jnp.arange(16)[:, None]
    j = jnp.arange(HID)[None, :]
    return jnp.where((j // DK) == i, 1.0, 0.0).astype(jnp.float32)


def _block_diag(A):
    r = A.shape[0]
    out = jnp.zeros((r, H, DK, H, DK), A.dtype)
    idx = jnp.arange(H)
    out = out.at[:, idx, :, idx, :].set(jnp.moveaxis(A, 1, 0))
    return out.reshape(r, HID, HID)


_SC_PARAMS = pltpu.CompilerParams(
    needs_layout_passes=False, use_tc_tiling_on_sc=False)


@functools.cache
def _mesh():
    return plsc.VectorSubcoreMesh(core_axis_name="c", subcore_axis_name="s",
                                  num_cores=NC, num_subcores=NS)


def _worker_id():
    return lax.axis_index("c") * NS + lax.axis_index("s")


def _nj(nchunk):
    w = _worker_id()
    extra = nchunk - (nchunk // NW) * NW
    return w, jnp.where(w < extra, nchunk // NW + 1, nchunk // NW)


def _iota16():
    return lax.iota(jnp.int32, 16)


def _pipeline(nchunk, load_issue, wait_compute):
    w, nj = _nj(nchunk)

    def ci(i):
        return w + NW * i

    load_issue(ci(0), 0)

    def pair(j, carry):
        i1 = 2 * j + 1
        i2 = 2 * j + 2

        @pl.when(i1 < nj)
        def _():
            load_issue(ci(i1), 1)
        wait_compute(ci(2 * j), 0)

        @pl.when(i2 < nj)
        def _():
            load_issue(ci(i2), 0)

        @pl.when(i1 < nj)
        def _():
            wait_compute(ci(i1), 1)
        return carry

    lax.fori_loop(0, (nj + 1) // 2, pair, 0)


def _pass1a_body(first, *refs):
    if first:
        (qn, kall, rk2, epack, ntr,
         att_o, eout_o,
         ntv, eb0, eb1, ob0, ob1,
         q0, k0, rk0, q1, k1, rk1, attT,
         sq0, sk0, sr0, sq1, sk1, sr1) = refs
        ebufs = (eb0, eb1)
        obufs = (ob0, ob1)
        pltpu.sync_copy(ntr, ntv.at[pl.ds(0, N)])
    else:
        (qn, kall, rk2, epack,
         att_o,
         eb0, eb1,
         q0, k0, rk0, q1, k1, rk1, attT,
         sq0, sk0, sr0, sq1, sk1, sr1) = refs
        ebufs = (eb0, eb1)
        obufs = ebufs
    qb = (q0, q1)
    kb = (k0, k1)
    rkb = (rk0, rk1)
    sems = ((sq0, sk0, sr0), (sq1, sk1, sr1))

    def load_issue(c, b):
        base = c * CB
        eb = ebufs[b]
        ob = obufs[b]
        pltpu.sync_copy(epack.at[:, pl.ds(base, CB)], eb)
        if first:
            for g in range(CB // 16):
                sl = pl.ds(g * 16, 16)
                s16 = eb[0, sl]
                d16 = eb[1, sl]
                e16 = eb[2, sl]
                r16 = eb[3, sl]
                tj = plsc.load_gather(ntv, [s16])
                ob[0, sl] = d16
                ob[1, sl] = s16 * R + r16
                ob[2, sl] = (e16 * T + tj) * R + r16
                ob[3, sl] = d16
            pltpu.sync_copy(ob, eout_o.at[:, pl.ds(base, CB)])
        pltpu.async_copy(qn.at[ob.at[0]], qb[b], sems[b][0])
        pltpu.async_copy(kall.at[ob.at[1]], kb[b], sems[b][1])
        pltpu.async_copy(rk2.at[ob.at[2]], rkb[b], sems[b][2])

    def wait_compute(c, b):
        base = c * CB
        ob = obufs[b]
        pltpu.make_async_copy(qn.at[ob.at[0]], qb[b], sems[b][0]).wait()
        pltpu.make_async_copy(kall.at[ob.at[1]], kb[b], sems[b][1]).wait()
        pltpu.make_async_copy(rk2.at[ob.at[2]], rkb[b], sems[b][2]).wait()

        lane15 = _iota16() == 15

        @plsc.parallel_loop(0, CB, 1, unroll=4)
        def edge(e):
            e16 = jnp.full((16,), e, jnp.int32)
            for h in range(H):
                sl = pl.ds(h * DK, DK)
                prod = qb[b][e, sl] * (kb[b][e, sl] + rkb[b][e, sl])
                tot = plsc.cumsum(prod)
                plsc.store_scatter(attT,
                                   [jnp.full((16,), h, jnp.int32), e16],
                                   tot, mask=lane15)
        pltpu.sync_copy(attT, att_o.at[:, pl.ds(base, CB)])

    _pipeline(E // CB, load_issue, wait_compute)


def _row_bufs(cb):
    return [pltpu.VMEM((cb, HID), jnp.float32)] * 3


def _pass1a_first(Qn, Kall, RK2, epack, nt):
    scratch = (
        [pltpu.VMEM((10112,), jnp.int32)]
        + [pltpu.VMEM((4, CB), jnp.int32)] * 4
        + _row_bufs(CB) + _row_bufs(CB)
        + [pltpu.VMEM((H, CB), jnp.float32)]
        + [pltpu.SemaphoreType.DMA] * 6
    )
    fn = pl.kernel(
        functools.partial(_pass1a_body, True),
        out_type=(_f32(H, E), _i32(4, E)),
        mesh=_mesh(),
        scratch_types=scratch,
        compiler_params=_SC_PARAMS,
    )
    return fn(Qn, Kall, RK2, epack, nt)


def _pass1a_rest(Qn, Kall, RK2, eout):
    scratch = (
        [pltpu.VMEM((4, CB), jnp.int32)] * 2
        + _row_bufs(CB) + _row_bufs(CB)
        + [pltpu.VMEM((H, CB), jnp.float32)]
        + [pltpu.SemaphoreType.DMA] * 6
    )
    fn = pl.kernel(
        functools.partial(_pass1a_body, False),
        out_type=_f32(H, E),
        mesh=_mesh(),
        scratch_types=scratch,
        compiler_params=_SC_PARAMS,
    )
    return fn(Qn, Kall, RK2, eout)


def _pass1b_body(attr, eout, mpart_o, dstv, attT, mloc):
    neg = jnp.full((16,), NEG, jnp.float32)

    def init(i, c):
        mloc[pl.ds(i * 16, 16)] = neg
        return c
    lax.fori_loop(0, (N * H) // 16, init, 0)

    w, nj = _nj(E // CB1B)

    def chunk(i, carry):
        base = (w + NW * i) * CB1B
        pltpu.sync_copy(eout.at[0, pl.ds(base, CB1B)], dstv)
        pltpu.sync_copy(attr.at[:, pl.ds(base, CB1B)], attT)

        def grp(g, carry2):
            dst16 = dstv[pl.ds(g * 16, 16)]
            for h in range(H):
                idx = dst16 + h * N
                val = attT[h, pl.ds(g * 16, 16)]
                cur = plsc.load_gather(mloc, [idx])
                msk = val > cur

                def cond(mm):
                    return jnp.any(mm)

                def body(mm):
                    plsc.store_scatter(mloc, [idx], val, mask=mm)
                    cur2 = plsc.load_gather(mloc, [idx])
                    return mm & (val > cur2)
                lax.while_loop(cond, body, msk)
            return carry2
        lax.fori_loop(0, CB1B // 16, grp, 0)
        return carry
    lax.fori_loop(0, nj, chunk, 0)
    pltpu.sync_copy(mloc, mpart_o.at[_worker_id()])


def _pass1b(att, eout):
    scratch = [
        pltpu.VMEM((CB1B,), jnp.int32),
        pltpu.VMEM((H, CB1B), jnp.float32),
        pltpu.VMEM((N * H,), jnp.float32),
    ]
    fn = pl.kernel(
        _pass1b_body,
        out_type=_f32(NW, N * H),
        mesh=_mesh(),
        scratch_types=scratch,
        compiler_params=_SC_PARAMS,
    )
    return fn(att, eout)


ROWS_PER_TILE = N // NS
ZCH = 5


def _pass2_body(attr, mr, eout, vall, rv2,
                aggp_o, sp_o,
                eb0, eb1, m0, m1, v0, rv0, v1, rv1, a0, a1,
                msgrows, wT, wrows, zbuf, zs, agg_s, s_s,
                sm0, sv0, sr0, sm1, sv1, sr1):
    cid = lax.axis_index("c")
    sid = lax.axis_index("s")
    r0 = sid * ROWS_PER_TILE
    ebufs = (eb0, eb1)
    mb = (m0, m1)
    vb = (v0, v1)
    rvb = (rv0, rv1)
    ab = (a0, a1)
    sems = ((sm0, sv0, sr0), (sm1, sv1, sr1))

    z16 = jnp.zeros((16,), jnp.float32)

    def zinit(i, c):
        for cc in range(HID // 16):
            zbuf[i, pl.ds(cc * 16, 16)] = z16
        zs[i, :] = z16
        return c
    lax.fori_loop(0, ZCH, zinit, 0)

    def winit(i, c):
        wrows[i, :] = z16
        return c
    lax.fori_loop(0, CB2, winit, 0)

    for j in range(ROWS_PER_TILE // ZCH):
        pltpu.sync_copy(zbuf, agg_s.at[pl.ds(r0 + j * ZCH, ZCH)])
        pltpu.sync_copy(zs, s_s.at[pl.ds(r0 + j * ZCH, ZCH)])
    plsc.subcore_barrier()

    def load_issue(c, b):
        base = c * CB2
        eb = ebufs[b]
        pltpu.sync_copy(eout.at[:, pl.ds(base, CB2)], eb)
        pltpu.sync_copy(attr.at[:, pl.ds(base, CB2)], ab[b])
        pltpu.async_copy(mr.at[eb.at[0]], mb[b], sems[b][0])
        pltpu.async_copy(vall.at[eb.at[1]], vb[b], sems[b][1])
        pltpu.async_copy(rv2.at[eb.at[2]], rvb[b], sems[b][2])

    def wait_compute(c, b):
        eb = ebufs[b]
        pltpu.make_async_copy(mr.at[eb.at[0]], mb[b], sems[b][0]).wait()
        pltpu.make_async_copy(vall.at[eb.at[1]], vb[b], sems[b][1]).wait()
        pltpu.make_async_copy(rv2.at[eb.at[2]], rvb[b], sems[b][2]).wait()

        for g in range(CB2 // 16):
            erow = g * 16 + _iota16()
            for h in range(H):
                hcol = jnp.full((16,), h, jnp.int32)
                m16 = plsc.load_gather(mb[b], [erow, hcol])
                a16 = ab[b][h, pl.ds(g * 16, 16)]
                w16 = jnp.exp(a16 - m16)
                wT[h, pl.ds(g * 16, 16)] = w16
                plsc.store_scatter(wrows, [erow, hcol], w16)

        iot = _iota16()

        def edge(e, carry):
            wrow = wrows[e, :]
            for h in range(H):
                c1 = plsc.cumsum(jnp.where(iot == h, wrow, 0.0))
                wb = jnp.maximum(c1, lax.rev(c1, (0,)))
                v16 = vb[b][e, pl.ds(h * DK, DK)]
                rv16 = rvb[b][e, pl.ds(h * DK, DK)]
                msgrows[e, pl.ds(h * DK, DK)] = wb * (v16 + rv16)
            return carry
        lax.fori_loop(0, CB2, edge, 0)

        pltpu.sync_copy(msgrows, agg_s.at[eb.at[0]], add=True)
        pltpu.sync_copy(wrows, s_s.at[eb.at[0]], add=True)

    _pipeline(E // CB2, load_issue, wait_compute)
    plsc.subcore_barrier()

    for j in range(ROWS_PER_TILE // ZCH):
        pltpu.sync_copy(agg_s.at[pl.ds(r0 + j * ZCH, ZCH)], zbuf)
        pltpu.sync_copy(zbuf, aggp_o.at[cid, pl.ds(r0 + j * ZCH, ZCH)])
        pltpu.sync_copy(s_s.at[pl.ds(r0 + j * ZCH, ZCH)], zs)
        pltpu.sync_copy(zs, sp_o.at[cid, pl.ds(r0 + j * ZCH, ZCH)])


def _pass2(att, m2, eout, Vall, RV2):
    scratch = [
        pltpu.VMEM((4, CB2), jnp.int32),
        pltpu.VMEM((4, CB2), jnp.int32),
        pltpu.VMEM((CB2, 16), jnp.float32),
        pltpu.VMEM((CB2, 16), jnp.float32),
        pltpu.VMEM((CB2, HID), jnp.float32),
        pltpu.VMEM((CB2, HID), jnp.float32),
        pltpu.VMEM((CB2, HID), jnp.float32),
        pltpu.VMEM((CB2, HID), jnp.float32),
        pltpu.VMEM((H, CB2), jnp.float32),
        pltpu.VMEM((H, CB2), jnp.float32),
        pltpu.VMEM((CB2, HID), jnp.float32),
        pltpu.VMEM((H, CB2), jnp.float32),
        pltpu.VMEM((CB2, 16), jnp.float32),
        pltpu.VMEM((ZCH, HID), jnp.float32),
        pltpu.VMEM((ZCH, 16), jnp.float32),
        pltpu.VMEM_SHARED((N, HID), jnp.float32),
        pltpu.VMEM_SHARED((N, 16), jnp.float32),
    ] + [pltpu.SemaphoreType.DMA] * 6
    fn = pl.kernel(
        _pass2_body,
        out_type=(_f32(NC, N, HID), _f32(NC, N, 16)),
        mesh=_mesh(),
        scratch_types=scratch,
        compiler_params=_SC_PARAMS,
    )
    return fn(att, m2, eout, Vall, RV2)


def kernel(node_feature, adapt_W, adapt_b, k_W, k_b, q_W, q_b, v_W, v_b,
           a_W, a_b, rel_pri, rel_att, rel_msg, skip, rte_W, rte_b,
           rte_emb, node_type, edge_time, edge_index, edge_type):
    nt = node_type.astype(jnp.int32)
    epack = jnp.stack([edge_index[0].astype(jnp.int32),
                       edge_index[1].astype(jnp.int32),
                       edge_time.astype(jnp.int32),
                       edge_type.astype(jnp.int32)])
    oh = jax.nn.one_hot(nt, T, dtype=jnp.float32)

    x = _adapt(node_feature, oh, adapt_W, adapt_b)
    eout = None
    for l in range(L):
        scale = rel_pri[l] / SQRT_DK
        BDk = _block_diag(rel_att[l] * scale[..., None, None])
        BDv = _block_diag(rel_msg[l])
        Qn, Kall, Vall = _proj(x, oh, k_W[l], k_b[l], q_W[l], q_b[l],
                               v_W[l], v_b[l], BDk, BDv)
        RK2, RV2 = _rk2(rte_emb, rte_W[l], rte_b[l], k_W[l], v_W[l], BDk, BDv)
        Kall = Kall.reshape(N * R, HID)
        Vall = Vall.reshape(N * R, HID)
        RK2 = RK2.reshape(ML * T * R, HID)
        RV2 = RV2.reshape(ML * T * R, HID)
        if l == 0:
            att, eout = _pass1a_first(Qn, Kall, RK2, epack, nt)
        else:
            att = _pass1a_rest(Qn, Kall, RK2, eout)
        mpart = _pass1b(att, eout)
        m = _merge(mpart)
        m2 = jnp.pad(m.reshape(H, N).T, ((0, 0), (0, 16 - H)))
        aggp, sp = _pass2(att, m2, eout, Vall, RV2)
        x = _combine(aggp, sp, x, oh, a_W[l], a_b[l], skip[l],
                     final=(l == L - 1))
    return x


# speedup vs baseline: 10.8829x; 1.0172x over previous
import functools
import math

import jax
import jax.numpy as jnp
from jax import lax
from jax.experimental import pallas as pl
from jax.experimental.pallas import tpu as pltpu
from jax.experimental.pallas import tpu_sc as plsc

N = 10000
E = 320000
IN_DIM = 128
HID = 128
T = 4
R = 8
H = 8
DK = HID // H
L = 2
ML = 240
SQRT_DK = math.sqrt(DK)

NC = 2
NS = 16
NW = NC * NS
CB = 128
CB2 = 32
CB1B = 512
NB = 1000
GRID = N // NB
NEG = -3.0e38


def _f32(*shape):
    return jax.ShapeDtypeStruct(shape, jnp.float32)


def _i32(*shape):
    return jax.ShapeDtypeStruct(shape, jnp.int32)


def _adapt_body(nf, oh, W, b, o):
    x = nf[:]
    acc = jnp.zeros((NB, HID), jnp.float32)
    for t in range(T):
        y = jnp.dot(x, W[t], preferred_element_type=jnp.float32) + b[t][None, :]
        acc = acc + oh[:, t][:, None] * y
    o[:] = jnp.tanh(acc)


def _adapt(nf, oh, W, b):
    return pl.pallas_call(
        _adapt_body,
        grid=(GRID,),
        in_specs=[
            pl.BlockSpec((NB, IN_DIM), lambda i: (i, 0)),
            pl.BlockSpec((NB, T), lambda i: (i, 0)),
            pl.BlockSpec((T, IN_DIM, HID), lambda i: (0, 0, 0)),
            pl.BlockSpec((T, HID), lambda i: (0, 0)),
        ],
        out_specs=pl.BlockSpec((NB, HID), lambda i: (i, 0)),
        out_shape=_f32(N, HID),
    )(nf, oh, W, b)


def _ptl_block(x, oh, W, b):
    acc = jnp.zeros((x.shape[0], HID), jnp.float32)
    for t in range(T):
        y = jnp.dot(x, W[t], preferred_element_type=jnp.float32) + b[t][None, :]
        acc = acc + oh[:, t][:, None] * y
    return acc


def _proj_body(x, oh, kW, kb, qW, qb, vW, vb, BDk, BDv, qo, ko, vo):
    xb = x[:]
    ohb = oh[:]
    Kn = _ptl_block(xb, ohb, kW, kb)
    qo[:] = _ptl_block(xb, ohb, qW, qb)
    Vn = _ptl_block(xb, ohb, vW, vb)
    for r in range(R):
        ko[:, r * HID:(r + 1) * HID] = jnp.dot(
            Kn, BDk[r], preferred_element_type=jnp.float32)
        vo[:, r * HID:(r + 1) * HID] = jnp.dot(
            Vn, BDv[r], preferred_element_type=jnp.float32)


def _proj(x, oh, kW, kb, qW, qb, vW, vb, BDk, BDv):
    wspec3 = pl.BlockSpec((T, HID, HID), lambda i: (0, 0, 0))
    wspec2 = pl.BlockSpec((T, HID), lambda i: (0, 0))
    bdspec = pl.BlockSpec((R, HID, HID), lambda i: (0, 0, 0))
    return pl.pallas_call(
        _proj_body,
        grid=(GRID,),
        in_specs=[
            pl.BlockSpec((NB, HID), lambda i: (i, 0)),
            pl.BlockSpec((NB, T), lambda i: (i, 0)),
            wspec3, wspec2, wspec3, wspec2, wspec3, wspec2, bdspec, bdspec,
        ],
        out_specs=[
            pl.BlockSpec((NB, HID), lambda i: (i, 0)),
            pl.BlockSpec((NB, R * HID), lambda i: (i, 0)),
            pl.BlockSpec((NB, R * HID), lambda i: (i, 0)),
        ],
        out_shape=[_f32(N, HID), _f32(N, R * HID), _f32(N, R * HID)],
    )(x, oh, kW, kb, qW, qb, vW, vb, BDk, BDv)


def _rk2_body(emb, rteW, rteb, kW, vW, BDk, BDv, ko, vo):
    r_vec = jnp.dot(emb[:], rteW[:],
                    preferred_element_type=jnp.float32) + rteb[:]
    for t in range(T):
        RKt = jnp.dot(r_vec, kW[t], preferred_element_type=jnp.float32)
        RVt = jnp.dot(r_vec, vW[t], preferred_element_type=jnp.float32)
        for r in range(R):
            c = (t * R + r) * HID
            ko[:, c:c + HID] = jnp.dot(RKt, BDk[r],
                                       preferred_element_type=jnp.float32)
            vo[:, c:c + HID] = jnp.dot(RVt, BDv[r],
                                       preferred_element_type=jnp.float32)


def _rk2(emb, rteW, rteb, kW, vW, BDk, BDv):
    return pl.pallas_call(
        _rk2_body,
        out_shape=[_f32(ML, T * R * HID), _f32(ML, T * R * HID)],
    )(emb, rteW, rteb.reshape(1, HID), kW, vW, BDk, BDv)


def _merge_body(p, o):
    o[:] = jnp.max(p[:], axis=0)


def _merge(mpart):
    return pl.pallas_call(
        _merge_body,
        out_shape=_f32(N * H),
    )(mpart)


def _combine_body(final, aggp, sp, x, oh, aW, ab, skp, REP, o):
    agg = aggp[0] + aggp[1]
    s16 = sp[0] + sp[1]
    denom = jnp.dot(s16, REP[:], preferred_element_type=jnp.float32) + 1e-16
    aggr = agg / denom
    aggr = 0.5 * aggr * (1.0 + lax.erf(aggr / math.sqrt(2.0)))
    trans = _ptl_block(aggr, oh[:], aW, ab)
    alphas = jax.nn.sigmoid(skp[:])
    alpha = jnp.sum(oh[:] * alphas, axis=1, keepdims=True)
    y = trans * alpha + x[:] * (1.0 - alpha)
    if final:
        y = y / jnp.sqrt(jnp.sum(y * y, axis=-1, keepdims=True))
    o[:] = y


def _combine(aggp, sp, x, oh, aW, ab, skp, final):
    return pl.pallas_call(
        functools.partial(_combine_body, final),
        grid=(GRID,),
        in_specs=[
            pl.BlockSpec((NC, NB, HID), lambda i: (0, i, 0)),
            pl.BlockSpec((NC, NB, 16), lambda i: (0, i, 0)),
            pl.BlockSpec((NB, HID), lambda i: (i, 0)),
            pl.BlockSpec((NB, T), lambda i: (i, 0)),
            pl.BlockSpec((T, HID, HID), lambda i: (0, 0, 0)),
            pl.BlockSpec((T, HID), lambda i: (0, 0)),
            pl.BlockSpec((1, T), lambda i: (0, 0)),
            pl.BlockSpec((16, HID), lambda i: (0, 0)),
        ],
        out_specs=pl.BlockSpec((NB, HID), lambda i: (i, 0)),
        out_shape=_f32(N, HID),
    )(aggp, sp, x, oh, aW, ab, skp.reshape(1, T), _rep_matrix())


def _rep_matrix():
    i = jnp.arange(16)[:, None]
    j = jnp.arange(HID)[None, :]
    return jnp.where((j // DK) == i, 1.0, 0.0).astype(jnp.float32)


def _block_diag(A):
    r = A.shape[0]
    out = jnp.zeros((r, H, DK, H, DK), A.dtype)
    idx = jnp.arange(H)
    out = out.at[:, idx, :, idx, :].set(jnp.moveaxis(A, 1, 0))
    return out.reshape(r, HID, HID)


_SC_PARAMS = pltpu.CompilerParams(
    needs_layout_passes=False, use_tc_tiling_on_sc=False)


@functools.cache
def _mesh():
    return plsc.VectorSubcoreMesh(core_axis_name="c", subcore_axis_name="s",
                                  num_cores=NC, num_subcores=NS)


def _worker_id():
    return lax.axis_index("c") * NS + lax.axis_index("s")


def _nj(nchunk):
    w = _worker_id()
    extra = nchunk - (nchunk // NW) * NW
    return w, jnp.where(w < extra, nchunk // NW + 1, nchunk // NW)


def _iota16():
    return lax.iota(jnp.int32, 16)


def _pipeline(nchunk, load_issue, wait_compute):
    w, nj = _nj(nchunk)

    def ci(i):
        return w + NW * i

    load_issue(ci(0), 0)

    def pair(j, carry):
        i1 = 2 * j + 1
        i2 = 2 * j + 2

        @pl.when(i1 < nj)
        def _():
            load_issue(ci(i1), 1)
        wait_compute(ci(2 * j), 0)

        @pl.when(i2 < nj)
        def _():
            load_issue(ci(i2), 0)

        @pl.when(i1 < nj)
        def _():
            wait_compute(ci(i1), 1)
        return carry

    lax.fori_loop(0, (nj + 1) // 2, pair, 0)


def _pass1a_body(first, *refs):
    if first:
        (qn, kall, rk2, epack, ntr,
         att_o, eout_o,
         ntv, eb0, eb1, ob0, ob1,
         q0, k0, rk0, q1, k1, rk1, attT,
         sq0, sk0, sr0, sq1, sk1, sr1) = refs
        ebufs = (eb0, eb1)
        obufs = (ob0, ob1)
        pltpu.sync_copy(ntr, ntv.at[pl.ds(0, N)])
    else:
        (qn, kall, rk2, epack,
         att_o,
         eb0, eb1,
         q0, k0, rk0, q1, k1, rk1, attT,
         sq0, sk0, sr0, sq1, sk1, sr1) = refs
        ebufs = (eb0, eb1)
        obufs = ebufs
    qb = (q0, q1)
    kb = (k0, k1)
    rkb = (rk0, rk1)
    sems = ((sq0, sk0, sr0), (sq1, sk1, sr1))

    def load_issue(c, b):
        base = c * CB
        eb = ebufs[b]
        ob = obufs[b]
        pltpu.sync_copy(epack.at[:, pl.ds(base, CB)], eb)
        if first:
            for g in range(CB // 16):
                sl = pl.ds(g * 16, 16)
                s16 = eb[0, sl]
                d16 = eb[1, sl]
                e16 = eb[2, sl]
                r16 = eb[3, sl]
                tj = plsc.load_gather(ntv, [s16])
                ob[0, sl] = d16
                ob[1, sl] = s16 * R + r16
                ob[2, sl] = (e16 * T + tj) * R + r16
                ob[3, sl] = d16
            pltpu.sync_copy(ob, eout_o.at[:, pl.ds(base, CB)])
        pltpu.async_copy(qn.at[ob.at[0]], qb[b], sems[b][0])
        pltpu.async_copy(kall.at[ob.at[1]], kb[b], sems[b][1])
        pltpu.async_copy(rk2.at[ob.at[2]], rkb[b], sems[b][2])

    def wait_compute(c, b):
        base = c * CB
        ob = obufs[b]
        pltpu.make_async_copy(qn.at[ob.at[0]], qb[b], sems[b][0]).wait()
        pltpu.make_async_copy(kall.at[ob.at[1]], kb[b], sems[b][1]).wait()
        pltpu.make_async_copy(rk2.at[ob.at[2]], rkb[b], sems[b][2]).wait()

        lane15 = _iota16() == 15

        @plsc.parallel_loop(0, CB, 1, unroll=4)
        def edge(e):
            e16 = jnp.full((16,), e, jnp.int32)
            for h in range(H):
                sl = pl.ds(h * DK, DK)
                prod = qb[b][e, sl] * (kb[b][e, sl] + rkb[b][e, sl])
                tot = plsc.cumsum(prod)
                plsc.store_scatter(attT,
                                   [jnp.full((16,), h, jnp.int32), e16],
                                   tot, mask=lane15)
        pltpu.sync_copy(attT, att_o.at[:, pl.ds(base, CB)])

    _pipeline(E // CB, load_issue, wait_compute)


def _row_bufs(cb):
    return [pltpu.VMEM((cb, HID), jnp.float32)] * 3


def _pass1a_first(Qn, Kall, RK2, epack, nt):
    scratch = (
        [pltpu.VMEM((10112,), jnp.int32)]
        + [pltpu.VMEM((4, CB), jnp.int32)] * 4
        + _row_bufs(CB) + _row_bufs(CB)
        + [pltpu.VMEM((H, CB), jnp.float32)]
        + [pltpu.SemaphoreType.DMA] * 6
    )
    fn = pl.kernel(
        functools.partial(_pass1a_body, True),
        out_type=(_f32(H, E), _i32(4, E)),
        mesh=_mesh(),
        scratch_types=scratch,
        compiler_params=_SC_PARAMS,
    )
    return fn(Qn, Kall, RK2, epack, nt)


def _pass1a_rest(Qn, Kall, RK2, eout):
    scratch = (
        [pltpu.VMEM((4, CB), jnp.int32)] * 2
        + _row_bufs(CB) + _row_bufs(CB)
        + [pltpu.VMEM((H, CB), jnp.float32)]
        + [pltpu.SemaphoreType.DMA] * 6
    )
    fn = pl.kernel(
        functools.partial(_pass1a_body, False),
        out_type=_f32(H, E),
        mesh=_mesh(),
        scratch_types=scratch,
        compiler_params=_SC_PARAMS,
    )
    return fn(Qn, Kall, RK2, eout)


def _pass1b_body(attr, eout, mpart_o, dstv, attT, mloc):
    neg = jnp.full((16,), NEG, jnp.float32)

    def init(i, c):
        mloc[pl.ds(i * 16, 16)] = neg
        return c
    lax.fori_loop(0, (N * H) // 16, init, 0)

    w, nj = _nj(E // CB1B)

    def chunk(i, carry):
        base = (w + NW * i) * CB1B
        pltpu.sync_copy(eout.at[0, pl.ds(base, CB1B)], dstv)
        pltpu.sync_copy(attr.at[:, pl.ds(base, CB1B)], attT)

        def grp(g, carry2):
            dst16 = dstv[pl.ds(g * 16, 16)]
            for h in range(H):
                idx = dst16 + h * N
                val = attT[h, pl.ds(g * 16, 16)]
                cur = plsc.load_gather(mloc, [idx])
                msk = val > cur

                def cond(mm):
                    return jnp.any(mm)

                def body(mm):
                    plsc.store_scatter(mloc, [idx], val, mask=mm)
                    cur2 = plsc.load_gather(mloc, [idx])
                    return mm & (val > cur2)
                lax.while_loop(cond, body, msk)
            return carry2
        lax.fori_loop(0, CB1B // 16, grp, 0)
        return carry
    lax.fori_loop(0, nj, chunk, 0)
    pltpu.sync_copy(mloc, mpart_o.at[_worker_id()])


def _pass1b(att, eout):
    scratch = [
        pltpu.VMEM((CB1B,), jnp.int32),
        pltpu.VMEM((H, CB1B), jnp.float32),
        pltpu.VMEM((N * H,), jnp.float32),
    ]
    fn = pl.kernel(
        _pass1b_body,
        out_type=_f32(NW, N * H),
        mesh=_mesh(),
        scratch_types=scratch,
        compiler_params=_SC_PARAMS,
    )
    return fn(att, eout)


ROWS_PER_TILE = N // NS
ZCH = 5


def _pass2_body(attr, mr, eout, vall, rv2,
                aggp_o, sp_o,
                eb0, eb1, m0, m1, v0, rv0, v1, rv1, a0, a1,
                msgrows, wT, wrows, zbuf, zs, agg_s, s_s,
                sm0, sv0, sr0, sm1, sv1, sr1):
    cid = lax.axis_index("c")
    sid = lax.axis_index("s")
    r0 = sid * ROWS_PER_TILE
    ebufs = (eb0, eb1)
    mb = (m0, m1)
    vb = (v0, v1)
    rvb = (rv0, rv1)
    ab = (a0, a1)
    sems = ((sm0, sv0, sr0), (sm1, sv1, sr1))

    z16 = jnp.zeros((16,), jnp.float32)

    def zinit(i, c):
        for cc in range(HID // 16):
            zbuf[i, pl.ds(cc * 16, 16)] = z16
        zs[i, :] = z16
        return c
    lax.fori_loop(0, ZCH, zinit, 0)

    def winit(i, c):
        wrows[i, :] = z16
        return c
    lax.fori_loop(0, CB2, winit, 0)

    for j in range(ROWS_PER_TILE // ZCH):
        pltpu.sync_copy(zbuf, agg_s.at[pl.ds(r0 + j * ZCH, ZCH)])
        pltpu.sync_copy(zs, s_s.at[pl.ds(r0 + j * ZCH, ZCH)])
    plsc.subcore_barrier()

    def load_issue(c, b):
        base = c * CB2
        eb = ebufs[b]
        pltpu.sync_copy(eout.at[:, pl.ds(base, CB2)], eb)
        pltpu.sync_copy(attr.at[:, pl.ds(base, CB2)], ab[b])
        pltpu.async_copy(mr.at[eb.at[0]], mb[b], sems[b][0])
        pltpu.async_copy(vall.at[eb.at[1]], vb[b], sems[b][1])
        pltpu.async_copy(rv2.at[eb.at[2]], rvb[b], sems[b][2])

    def wait_compute(c, b):
        eb = ebufs[b]
        pltpu.make_async_copy(mr.at[eb.at[0]], mb[b], sems[b][0]).wait()
        pltpu.make_async_copy(vall.at[eb.at[1]], vb[b], sems[b][1]).wait()
        pltpu.make_async_copy(rv2.at[eb.at[2]], rvb[b], sems[b][2]).wait()

        for g in range(CB2 // 16):
            erow = g * 16 + _iota16()
            for h in range(H):
                hcol = jnp.full((16,), h, jnp.int32)
                m16 = plsc.load_gather(mb[b], [erow, hcol])
                a16 = ab[b][h, pl.ds(g * 16, 16)]
                w16 = jnp.exp(a16 - m16)
                wT[h, pl.ds(g * 16, 16)] = w16
                plsc.store_scatter(wrows, [erow, hcol], w16)

        iot = _iota16()

        @plsc.parallel_loop(0, CB2, 1, unroll=4)
        def edge(e):
            wrow = wrows[e, :]
            for h in range(H):
                c1 = plsc.cumsum(jnp.where(iot == h, wrow, 0.0))
                wb = jnp.maximum(c1, lax.rev(c1, (0,)))
                v16 = vb[b][e, pl.ds(h * DK, DK)]
                rv16 = rvb[b][e, pl.ds(h * DK, DK)]
                msgrows[e, pl.ds(h * DK, DK)] = wb * (v16 + rv16)

        pltpu.sync_copy(msgrows, agg_s.at[eb.at[0]], add=True)
        pltpu.sync_copy(wrows, s_s.at[eb.at[0]], add=True)

    _pipeline(E // CB2, load_issue, wait_compute)
    plsc.subcore_barrier()

    for j in range(ROWS_PER_TILE // ZCH):
        pltpu.sync_copy(agg_s.at[pl.ds(r0 + j * ZCH, ZCH)], zbuf)
        pltpu.sync_copy(zbuf, aggp_o.at[cid, pl.ds(r0 + j * ZCH, ZCH)])
        pltpu.sync_copy(s_s.at[pl.ds(r0 + j * ZCH, ZCH)], zs)
        pltpu.sync_copy(zs, sp_o.at[cid, pl.ds(r0 + j * ZCH, ZCH)])


def _pass2(att, m2, eout, Vall, RV2):
    scratch = [
        pltpu.VMEM((4, CB2), jnp.int32),
        pltpu.VMEM((4, CB2), jnp.int32),
        pltpu.VMEM((CB2, 16), jnp.float32),
        pltpu.VMEM((CB2, 16), jnp.float32),
        pltpu.VMEM((CB2, HID), jnp.float32),
        pltpu.VMEM((CB2, HID), jnp.float32),
        pltpu.VMEM((CB2, HID), jnp.float32),
        pltpu.VMEM((CB2, HID), jnp.float32),
        pltpu.VMEM((H, CB2), jnp.float32),
        pltpu.VMEM((H, CB2), jnp.float32),
        pltpu.VMEM((CB2, HID), jnp.float32),
        pltpu.VMEM((H, CB2), jnp.float32),
        pltpu.VMEM((CB2, 16), jnp.float32),
        pltpu.VMEM((ZCH, HID), jnp.float32),
        pltpu.VMEM((ZCH, 16), jnp.float32),
        pltpu.VMEM_SHARED((N, HID), jnp.float32),
        pltpu.VMEM_SHARED((N, 16), jnp.float32),
    ] + [pltpu.SemaphoreType.DMA] * 6
    fn = pl.kernel(
        _pass2_body,
        out_type=(_f32(NC, N, HID), _f32(NC, N, 16)),
        mesh=_mesh(),
        scratch_types=scratch,
        compiler_params=_SC_PARAMS,
    )
    return fn(att, m2, eout, Vall, RV2)


def kernel(node_feature, adapt_W, adapt_b, k_W, k_b, q_W, q_b, v_W, v_b,
           a_W, a_b, rel_pri, rel_att, rel_msg, skip, rte_W, rte_b,
           rte_emb, node_type, edge_time, edge_index, edge_type):
    nt = node_type.astype(jnp.int32)
    epack = jnp.stack([edge_index[0].astype(jnp.int32),
                       edge_index[1].astype(jnp.int32),
                       edge_time.astype(jnp.int32),
                       edge_type.astype(jnp.int32)])
    oh = jax.nn.one_hot(nt, T, dtype=jnp.float32)

    x = _adapt(node_feature, oh, adapt_W, adapt_b)
    eout = None
    for l in range(L):
        scale = rel_pri[l] / SQRT_DK
        BDk = _block_diag(rel_att[l] * scale[..., None, None])
        BDv = _block_diag(rel_msg[l])
        Qn, Kall, Vall = _proj(x, oh, k_W[l], k_b[l], q_W[l], q_b[l],
                               v_W[l], v_b[l], BDk, BDv)
        RK2, RV2 = _rk2(rte_emb, rte_W[l], rte_b[l], k_W[l], v_W[l], BDk, BDv)
        Kall = Kall.reshape(N * R, HID)
        Vall = Vall.reshape(N * R, HID)
        RK2 = RK2.reshape(ML * T * R, HID)
        RV2 = RV2.reshape(ML * T * R, HID)
        if l == 0:
            att, eout = _pass1a_first(Qn, Kall, RK2, epack, nt)
        else:
            att = _pass1a_rest(Qn, Kall, RK2, eout)
        mpart = _pass1b(att, eout)
        m = _merge(mpart)
        m2 = jnp.pad(m.reshape(H, N).T, ((0, 0), (0, 16 - H)))
        aggp, sp = _pass2(att, m2, eout, Vall, RV2)
        x = _combine(aggp, sp, x, oh, a_W[l], a_b[l], skip[l],
                     final=(l == L - 1))
    return x
